# Initial kernel scaffold; baseline (speedup 1.0000x reference)
import jax
import jax.numpy as jnp
from jax.experimental import pallas as pl


def kernel(x, edge_index, W1, b1, W2, b2, Wo, bo):
    raise NotImplementedError("write your pallas kernel here")


# speedup vs baseline: 7.0098x; 7.0098x over previous
import functools

import jax
import jax.numpy as jnp
from jax import lax
from jax.experimental import pallas as pl
from jax.experimental.pallas import tpu as pltpu
from jax.experimental.pallas import tpu_sc as plsc

_NC = 2
_NS = 16
_LN = 16
_CKD = 40
_CKS = 80


def _pad_n(n):
    step = _NS * 8
    return ((n + step - 1) // step) * step


@functools.lru_cache(maxsize=None)
def _degree_sc(n, e, fw):
    epc = e // _NC
    ept = epc // _NS
    nchunks = ept // _CKD
    npad = _pad_n(n)
    rpt = npad // _NS

    mesh = plsc.VectorSubcoreMesh(
        core_axis_name="c", subcore_axis_name="s",
        num_cores=_NC, num_subcores=_NS)

    @functools.partial(
        pl.kernel,
        out_type=jax.ShapeDtypeStruct((_NC, npad, fw), jnp.float32),
        mesh=mesh,
        scratch_types=[
            pltpu.VMEM_SHARED((npad, fw), jnp.float32),
            pltpu.VMEM((_CKD, fw), jnp.float32),
            pltpu.VMEM((_CKD,), jnp.int32),
        ],
    )
    def deg_kernel(cols_hbm, zeros_hbm, ones_hbm, out_hbm, acc, ones_v, idx_v):
        c = lax.axis_index("c")
        s = lax.axis_index("s")
        pltpu.sync_copy(zeros_hbm.at[pl.ds(s * rpt, rpt)],
                        acc.at[pl.ds(s * rpt, rpt)])
        pltpu.sync_copy(ones_hbm, ones_v)
        plsc.subcore_barrier()
        base = c * epc + s * ept

        def body(i, carry):
            off = base + i * _CKD
            pltpu.sync_copy(cols_hbm.at[pl.ds(off, _CKD)], idx_v)
            pltpu.sync_copy(ones_v, acc.at[idx_v], add=True)
            return carry

        lax.fori_loop(0, nchunks, body, 0)
        plsc.subcore_barrier()
        pltpu.sync_copy(acc.at[pl.ds(s * rpt, rpt)],
                        out_hbm.at[c, pl.ds(s * rpt, rpt)])

    return deg_kernel


@functools.lru_cache(maxsize=None)
def _scatter_sc(n, e, f):
    ept = e // _NS
    nchunks = ept // _CKS
    npad = _pad_n(n)
    rpt = npad // _NS

    mesh = plsc.VectorSubcoreMesh(
        core_axis_name="c", subcore_axis_name="s",
        num_cores=_NC, num_subcores=_NS)

    @functools.partial(
        pl.kernel,
        out_type=jax.ShapeDtypeStruct((_NC, npad, f), jnp.float32),
        mesh=mesh,
        scratch_types=[
            pltpu.VMEM_SHARED((npad, f), jnp.float32),
            pltpu.VMEM((_CKS, f), jnp.float32),
            pltpu.VMEM((_CKS,), jnp.int32),
            pltpu.VMEM((_CKS,), jnp.int32),
            pltpu.VMEM((_CKS,), jnp.int32),
            pltpu.SemaphoreType.DMA,
        ],
    )
    def scat_kernel(rows_hbm, cols_hbm, g_hbm, zeros_hbm, out_hbm,
                    acc, buf, ridx, aidx, cidx, sem):
        c = lax.axis_index("c")
        s = lax.axis_index("s")
        roff = c * n
        pltpu.sync_copy(zeros_hbm.at[pl.ds(s * rpt, rpt)],
                        acc.at[pl.ds(s * rpt, rpt)])
        plsc.subcore_barrier()
        base = s * ept

        def body(i, carry):
            off = base + i * _CKS
            pltpu.sync_copy(rows_hbm.at[pl.ds(off, _CKS)], ridx)
            pltpu.sync_copy(cols_hbm.at[pl.ds(off, _CKS)], cidx)
            for gi in range(_CKS // _LN):
                sl = pl.ds(gi * _LN, _LN)
                aidx[sl] = ridx[sl] + roff
            pltpu.async_copy(g_hbm.at[aidx], buf, sem).wait()
            pltpu.sync_copy(buf, acc.at[cidx], add=True)
            return carry

        lax.fori_loop(0, nchunks, body, 0)
        plsc.subcore_barrier()
        pltpu.sync_copy(acc.at[pl.ds(s * rpt, rpt)],
                        out_hbm.at[c, pl.ds(s * rpt, rpt)])

    return scat_kernel


def _dot(a, b):
    return jnp.dot(a, b, precision=lax.Precision.HIGHEST,
                   preferred_element_type=jnp.float32)


def _split2(g):
    half = g.shape[1] // 2
    return jnp.concatenate([g[None, :, :half], g[None, :, half:]], axis=0)


def _tc_a_body(deg_ref, x_ref, w_ref, h_ref, g_ref, dinv_ref):
    deg = deg_ref[0, :, :_LN] + deg_ref[1, :, :_LN] + 1.0
    dinv = lax.rsqrt(deg)
    hv = _dot(x_ref[...], w_ref[...])
    g = hv * dinv[:, 0:1]
    h_ref[...] = hv
    g_ref[...] = _split2(g)
    dinv_ref[...] = dinv


@functools.lru_cache(maxsize=None)
def _tc_a(n, d, h, blk):
    return pl.pallas_call(
        _tc_a_body,
        grid=(n // blk,),
        in_specs=[
            pl.BlockSpec((_NC, blk, h // 2), lambda b: (0, b, 0)),
            pl.BlockSpec((blk, d), lambda b: (b, 0)),
            pl.BlockSpec((d, h), lambda b: (0, 0)),
        ],
        out_specs=[
            pl.BlockSpec((blk, h), lambda b: (b, 0)),
            pl.BlockSpec((2, blk, h // 2), lambda b: (0, b, 0)),
            pl.BlockSpec((blk, _LN), lambda b: (b, 0)),
        ],
        out_shape=[
            jax.ShapeDtypeStruct((n, h), jnp.float32),
            jax.ShapeDtypeStruct((2, n, h // 2), jnp.float32),
            jax.ShapeDtypeStruct((n, _LN), jnp.float32),
        ],
    )


def _tc_b_body(alo_ref, ahi_ref, h1_ref, dinv_ref, b1_ref, w2_ref,
               h2_ref, g_ref):
    d1 = dinv_ref[:, 0:1]
    acc = jnp.concatenate([alo_ref[...], ahi_ref[...]], axis=1)
    out1 = d1 * acc + (d1 * d1) * h1_ref[...] + b1_ref[...]
    m = jnp.maximum(out1, 0.0)
    h2 = _dot(m, w2_ref[...])
    g2 = h2 * d1
    h2_ref[...] = h2
    g_ref[...] = _split2(g2)


@functools.lru_cache(maxsize=None)
def _tc_b(n, h, blk):
    return pl.pallas_call(
        _tc_b_body,
        grid=(n // blk,),
        in_specs=[
            pl.BlockSpec((blk, h // 2), lambda b: (b, 0)),
            pl.BlockSpec((blk, h // 2), lambda b: (b, 0)),
            pl.BlockSpec((blk, h), lambda b: (b, 0)),
            pl.BlockSpec((blk, _LN), lambda b: (b, 0)),
            pl.BlockSpec((1, h), lambda b: (0, 0)),
            pl.BlockSpec((h, h), lambda b: (0, 0)),
        ],
        out_specs=[
            pl.BlockSpec((blk, h), lambda b: (b, 0)),
            pl.BlockSpec((2, blk, h // 2), lambda b: (0, b, 0)),
        ],
        out_shape=[
            jax.ShapeDtypeStruct((n, h), jnp.float32),
            jax.ShapeDtypeStruct((2, n, h // 2), jnp.float32),
        ],
    )


def _tc_c_body(alo_ref, ahi_ref, h2_ref, dinv_ref, b2_ref, wo_ref, bo_ref,
               out_ref):
    d1 = dinv_ref[:, 0:1]
    acc = jnp.concatenate([alo_ref[...], ahi_ref[...]], axis=1)
    out2 = d1 * acc + (d1 * d1) * h2_ref[...] + b2_ref[...]
    logits = _dot(out2, wo_ref[...]) + bo_ref[...]
    mx = jnp.max(logits, axis=1, keepdims=True)
    sh = logits - mx
    lse = jnp.log(jnp.sum(jnp.exp(sh), axis=1, keepdims=True))
    out_ref[...] = sh - lse


@functools.lru_cache(maxsize=None)
def _tc_c(n, h, cdim, blk):
    return pl.pallas_call(
        _tc_c_body,
        grid=(n // blk,),
        in_specs=[
            pl.BlockSpec((blk, h // 2), lambda b: (b, 0)),
            pl.BlockSpec((blk, h // 2), lambda b: (b, 0)),
            pl.BlockSpec((blk, h), lambda b: (b, 0)),
            pl.BlockSpec((blk, _LN), lambda b: (b, 0)),
            pl.BlockSpec((1, h), lambda b: (0, 0)),
            pl.BlockSpec((h, cdim), lambda b: (0, 0)),
            pl.BlockSpec((1, cdim), lambda b: (0, 0)),
        ],
        out_specs=pl.BlockSpec((blk, cdim), lambda b: (b, 0)),
        out_shape=jax.ShapeDtypeStruct((n, cdim), jnp.float32),
    )


def kernel(x, edge_index, W1, b1, W2, b2, Wo, bo):
    n, d = x.shape
    e = edge_index.shape[1]
    h = W1.shape[1]
    cdim = Wo.shape[1]
    f = h // 2
    blk = 400

    npad = _pad_n(n)
    rows = edge_index[0]
    cols = edge_index[1]
    ones_d = jnp.ones((_CKD, f), jnp.float32)
    zeros_f = jnp.zeros((npad, f), jnp.float32)

    degs = _degree_sc(n, e, f)(cols, zeros_f, ones_d)[:, :n]
    h1, g1, dinv = _tc_a(n, d, h, blk)(degs, x, W1)
    acc1 = _scatter_sc(n, e, f)(rows, cols, g1.reshape(2 * n, f), zeros_f)
    h2, g2 = _tc_b(n, h, blk)(
        acc1[0, :n], acc1[1, :n], h1, dinv, b1.reshape(1, -1), W2)
    acc2 = _scatter_sc(n, e, f)(rows, cols, g2.reshape(2 * n, f), zeros_f)
    return _tc_c(n, h, cdim, blk)(
        acc2[0, :n], acc2[1, :n], h2, dinv, b2.reshape(1, -1), Wo, bo.reshape(1, -1))


# speedup vs baseline: 7.8907x; 1.1257x over previous
import functools

import jax
import jax.numpy as jnp
from jax import lax
from jax.experimental import pallas as pl
from jax.experimental.pallas import tpu as pltpu
from jax.experimental.pallas import tpu_sc as plsc

_NC = 2
_NS = 16
_LN = 16
_CKD = 40
_CKS = 80


def _pad_n(n):
    step = _NS * 8
    return ((n + step - 1) // step) * step


@functools.lru_cache(maxsize=None)
def _degree_sc(n, e, fw):
    epc = e // _NC
    ept = epc // _NS
    nchunks = ept // _CKD
    npad = _pad_n(n)
    rpt = npad // _NS

    mesh = plsc.VectorSubcoreMesh(
        core_axis_name="c", subcore_axis_name="s",
        num_cores=_NC, num_subcores=_NS)

    @functools.partial(
        pl.kernel,
        out_type=jax.ShapeDtypeStruct((_NC, npad, fw), jnp.float32),
        mesh=mesh,
        scratch_types=[
            pltpu.VMEM_SHARED((npad, fw), jnp.float32),
            pltpu.VMEM((_CKD, fw), jnp.float32),
            pltpu.VMEM((_CKD,), jnp.int32),
        ],
    )
    def deg_kernel(cols_hbm, zeros_hbm, ones_hbm, out_hbm, acc, ones_v, idx_v):
        c = lax.axis_index("c")
        s = lax.axis_index("s")
        pltpu.sync_copy(zeros_hbm.at[pl.ds(s * rpt, rpt)],
                        acc.at[pl.ds(s * rpt, rpt)])
        pltpu.sync_copy(ones_hbm, ones_v)
        plsc.subcore_barrier()
        base = c * epc + s * ept

        def body(i, carry):
            off = base + i * _CKD
            pltpu.sync_copy(cols_hbm.at[pl.ds(off, _CKD)], idx_v)
            pltpu.sync_copy(ones_v, acc.at[idx_v], add=True)
            return carry

        lax.fori_loop(0, nchunks, body, 0)
        plsc.subcore_barrier()
        pltpu.sync_copy(acc.at[pl.ds(s * rpt, rpt)],
                        out_hbm.at[c, pl.ds(s * rpt, rpt)])

    return deg_kernel


@functools.lru_cache(maxsize=None)
def _scatter_sc(n, e, f):
    ept = e // _NS
    nchunks = ept // _CKS
    npairs = (nchunks - 1) // 2
    npad = _pad_n(n)
    rpt = npad // _NS

    mesh = plsc.VectorSubcoreMesh(
        core_axis_name="c", subcore_axis_name="s",
        num_cores=_NC, num_subcores=_NS)

    @functools.partial(
        pl.kernel,
        out_type=jax.ShapeDtypeStruct((_NC, npad, f), jnp.float32),
        mesh=mesh,
        scratch_types=[
            pltpu.VMEM_SHARED((npad, f), jnp.float32),
            pltpu.VMEM((_CKS,), jnp.int32),
            pltpu.VMEM((_CKS, f), jnp.float32),
            pltpu.VMEM((_CKS, f), jnp.float32),
            pltpu.VMEM((_CKS,), jnp.int32),
            pltpu.VMEM((_CKS,), jnp.int32),
            pltpu.VMEM((_CKS,), jnp.int32),
            pltpu.VMEM((_CKS,), jnp.int32),
            pltpu.SemaphoreType.DMA,
            pltpu.SemaphoreType.DMA,
        ],
    )
    def scat_kernel(rows_hbm, cols_hbm, g_hbm, zeros_hbm, out_hbm,
                    acc, ridx, b0, b1, a0, a1, c0, c1, semg, sems):
        bufs = (b0, b1)
        aidxs = (a0, a1)
        cidxs = (c0, c1)
        c = lax.axis_index("c")
        s = lax.axis_index("s")
        roff = c * n
        pltpu.sync_copy(zeros_hbm.at[pl.ds(s * rpt, rpt)],
                        acc.at[pl.ds(s * rpt, rpt)])
        plsc.subcore_barrier()
        base = s * ept

        def load_fire(i, sl):
            off = base + i * _CKS
            pltpu.sync_copy(rows_hbm.at[pl.ds(off, _CKS)], ridx)
            pltpu.sync_copy(cols_hbm.at[pl.ds(off, _CKS)], cidxs[sl])
            for gi in range(_CKS // _LN):
                lanes = pl.ds(gi * _LN, _LN)
                aidxs[sl][lanes] = ridx[lanes] + roff
            pltpu.async_copy(g_hbm.at[aidxs[sl]], bufs[sl], semg)

        def wait_gather(sl):
            pltpu.make_async_copy(g_hbm.at[aidxs[sl]], bufs[sl], semg).wait()

        def wait_scat(sl):
            pltpu.make_async_copy(zeros_hbm.at[pl.ds(0, _CKS)],
                                  bufs[sl], sems).wait()

        load_fire(0, 0)
        pltpu.sync_copy(zeros_hbm.at[pl.ds(0, _CKS)], bufs[1])
        pltpu.sync_copy(cols_hbm.at[pl.ds(base, _CKS)], cidxs[1])
        pltpu.async_copy(bufs[1], acc.at[cidxs[1]], sems, add=True)

        def step(i, cur, nxt):
            wait_gather(cur)
            wait_scat(nxt)
            load_fire(i + 1, nxt)
            pltpu.async_copy(bufs[cur], acc.at[cidxs[cur]], sems, add=True)

        def body(j, carry):
            step(2 * j, 0, 1)
            step(2 * j + 1, 1, 0)
            return carry

        lax.fori_loop(0, npairs, body, 0)
        wait_gather(0)
        wait_scat(1)
        pltpu.sync_copy(bufs[0], acc.at[cidxs[0]], add=True)
        plsc.subcore_barrier()
        pltpu.sync_copy(acc.at[pl.ds(s * rpt, rpt)],
                        out_hbm.at[c, pl.ds(s * rpt, rpt)])

    return scat_kernel


def _dot(a, b):
    return jnp.dot(a, b, precision=lax.Precision.HIGHEST,
                   preferred_element_type=jnp.float32)


def _split2(g):
    half = g.shape[1] // 2
    return jnp.concatenate([g[None, :, :half], g[None, :, half:]], axis=0)


def _tc_a_body(deg_ref, x_ref, w_ref, h_ref, g_ref, dinv_ref):
    deg = deg_ref[0, :, :_LN] + deg_ref[1, :, :_LN] + 1.0
    dinv = lax.rsqrt(deg)
    hv = _dot(x_ref[...], w_ref[...])
    g = hv * dinv[:, 0:1]
    h_ref[...] = hv
    g_ref[...] = _split2(g)
    dinv_ref[...] = dinv


@functools.lru_cache(maxsize=None)
def _tc_a(n, d, h, blk):
    return pl.pallas_call(
        _tc_a_body,
        grid=(n // blk,),
        in_specs=[
            pl.BlockSpec((_NC, blk, h // 2), lambda b: (0, b, 0)),
            pl.BlockSpec((blk, d), lambda b: (b, 0)),
            pl.BlockSpec((d, h), lambda b: (0, 0)),
        ],
        out_specs=[
            pl.BlockSpec((blk, h), lambda b: (b, 0)),
            pl.BlockSpec((2, blk, h // 2), lambda b: (0, b, 0)),
            pl.BlockSpec((blk, _LN), lambda b: (b, 0)),
        ],
        out_shape=[
            jax.ShapeDtypeStruct((n, h), jnp.float32),
            jax.ShapeDtypeStruct((2, n, h // 2), jnp.float32),
            jax.ShapeDtypeStruct((n, _LN), jnp.float32),
        ],
    )


def _tc_b_body(alo_ref, ahi_ref, h1_ref, dinv_ref, b1_ref, w2_ref,
               h2_ref, g_ref):
    d1 = dinv_ref[:, 0:1]
    acc = jnp.concatenate([alo_ref[...], ahi_ref[...]], axis=1)
    out1 = d1 * acc + (d1 * d1) * h1_ref[...] + b1_ref[...]
    m = jnp.maximum(out1, 0.0)
    h2 = _dot(m, w2_ref[...])
    g2 = h2 * d1
    h2_ref[...] = h2
    g_ref[...] = _split2(g2)


@functools.lru_cache(maxsize=None)
def _tc_b(n, h, blk):
    return pl.pallas_call(
        _tc_b_body,
        grid=(n // blk,),
        in_specs=[
            pl.BlockSpec((blk, h // 2), lambda b: (b, 0)),
            pl.BlockSpec((blk, h // 2), lambda b: (b, 0)),
            pl.BlockSpec((blk, h), lambda b: (b, 0)),
            pl.BlockSpec((blk, _LN), lambda b: (b, 0)),
            pl.BlockSpec((1, h), lambda b: (0, 0)),
            pl.BlockSpec((h, h), lambda b: (0, 0)),
        ],
        out_specs=[
            pl.BlockSpec((blk, h), lambda b: (b, 0)),
            pl.BlockSpec((2, blk, h // 2), lambda b: (0, b, 0)),
        ],
        out_shape=[
            jax.ShapeDtypeStruct((n, h), jnp.float32),
            jax.ShapeDtypeStruct((2, n, h // 2), jnp.float32),
        ],
    )


def _tc_c_body(alo_ref, ahi_ref, h2_ref, dinv_ref, b2_ref, wo_ref, bo_ref,
               out_ref):
    d1 = dinv_ref[:, 0:1]
    acc = jnp.concatenate([alo_ref[...], ahi_ref[...]], axis=1)
    out2 = d1 * acc + (d1 * d1) * h2_ref[...] + b2_ref[...]
    logits = _dot(out2, wo_ref[...]) + bo_ref[...]
    mx = jnp.max(logits, axis=1, keepdims=True)
    sh = logits - mx
    lse = jnp.log(jnp.sum(jnp.exp(sh), axis=1, keepdims=True))
    out_ref[...] = sh - lse


@functools.lru_cache(maxsize=None)
def _tc_c(n, h, cdim, blk):
    return pl.pallas_call(
        _tc_c_body,
        grid=(n // blk,),
        in_specs=[
            pl.BlockSpec((blk, h // 2), lambda b: (b, 0)),
            pl.BlockSpec((blk, h // 2), lambda b: (b, 0)),
            pl.BlockSpec((blk, h), lambda b: (b, 0)),
            pl.BlockSpec((blk, _LN), lambda b: (b, 0)),
            pl.BlockSpec((1, h), lambda b: (0, 0)),
            pl.BlockSpec((h, cdim), lambda b: (0, 0)),
            pl.BlockSpec((1, cdim), lambda b: (0, 0)),
        ],
        out_specs=pl.BlockSpec((blk, cdim), lambda b: (b, 0)),
        out_shape=jax.ShapeDtypeStruct((n, cdim), jnp.float32),
    )


def kernel(x, edge_index, W1, b1, W2, b2, Wo, bo):
    n, d = x.shape
    e = edge_index.shape[1]
    h = W1.shape[1]
    cdim = Wo.shape[1]
    f = h // 2
    blk = 400

    npad = _pad_n(n)
    rows = edge_index[0]
    cols = edge_index[1]
    ones_d = jnp.ones((_CKD, f), jnp.float32)
    zeros_f = jnp.zeros((npad, f), jnp.float32)

    degs = _degree_sc(n, e, f)(cols, zeros_f, ones_d)[:, :n]
    h1, g1, dinv = _tc_a(n, d, h, blk)(degs, x, W1)
    acc1 = _scatter_sc(n, e, f)(rows, cols, g1.reshape(2 * n, f), zeros_f)
    h2, g2 = _tc_b(n, h, blk)(
        acc1[0, :n], acc1[1, :n], h1, dinv, b1.reshape(1, -1), W2)
    acc2 = _scatter_sc(n, e, f)(rows, cols, g2.reshape(2 * n, f), zeros_f)
    return _tc_c(n, h, cdim, blk)(
        acc2[0, :n], acc2[1, :n], h2, dinv, b2.reshape(1, -1), Wo, bo.reshape(1, -1))


# speedup vs baseline: 7.9627x; 1.0091x over previous
import functools

import jax
import jax.numpy as jnp
from jax import lax
from jax.experimental import pallas as pl
from jax.experimental.pallas import tpu as pltpu
from jax.experimental.pallas import tpu_sc as plsc

_NC = 2
_NS = 16
_LN = 16
_CKD = 40
_CKS = 80


def _pad_n(n):
    step = _NS * 8
    return ((n + step - 1) // step) * step


@functools.lru_cache(maxsize=None)
def _degree_sc(n, e, fw):
    epc = e // _NC
    ept = epc // _NS
    nchunks = ept // _CKD
    npad = _pad_n(n)
    rpt = npad // _NS

    mesh = plsc.VectorSubcoreMesh(
        core_axis_name="c", subcore_axis_name="s",
        num_cores=_NC, num_subcores=_NS)

    @functools.partial(
        pl.kernel,
        out_type=jax.ShapeDtypeStruct((_NC, npad, fw), jnp.float32),
        mesh=mesh,
        scratch_types=[
            pltpu.VMEM_SHARED((npad, fw), jnp.float32),
            pltpu.VMEM((_CKD, fw), jnp.float32),
            pltpu.VMEM((_CKD,), jnp.int32),
        ],
    )
    def deg_kernel(cols_hbm, zeros_hbm, ones_hbm, out_hbm, acc, ones_v, idx_v):
        c = lax.axis_index("c")
        s = lax.axis_index("s")
        pltpu.sync_copy(zeros_hbm.at[pl.ds(s * rpt, rpt)],
                        acc.at[pl.ds(s * rpt, rpt)])
        pltpu.sync_copy(ones_hbm, ones_v)
        plsc.subcore_barrier()
        base = c * epc + s * ept

        def body(i, carry):
            off = base + i * _CKD
            pltpu.sync_copy(cols_hbm.at[pl.ds(off, _CKD)], idx_v)
            pltpu.sync_copy(ones_v, acc.at[idx_v], add=True)
            return carry

        lax.fori_loop(0, nchunks, body, 0)
        plsc.subcore_barrier()
        pltpu.sync_copy(acc.at[pl.ds(s * rpt, rpt)],
                        out_hbm.at[c, pl.ds(s * rpt, rpt)])

    return deg_kernel


@functools.lru_cache(maxsize=None)
def _scatter_sc(n, e, f):
    ept = e // _NS
    nchunks = ept // _CKS
    npairs = (nchunks - 1) // 2
    npad = _pad_n(n)
    rpt = npad // _NS

    mesh = plsc.VectorSubcoreMesh(
        core_axis_name="c", subcore_axis_name="s",
        num_cores=_NC, num_subcores=_NS)

    @functools.partial(
        pl.kernel,
        out_type=jax.ShapeDtypeStruct((_NC, npad, f), jnp.float32),
        mesh=mesh,
        scratch_types=[
            pltpu.VMEM_SHARED((npad, f), jnp.float32),
            pltpu.VMEM((_CKS,), jnp.int32),
            pltpu.VMEM((_CKS, f), jnp.float32),
            pltpu.VMEM((_CKS, f), jnp.float32),
            pltpu.VMEM((_CKS,), jnp.int32),
            pltpu.VMEM((_CKS,), jnp.int32),
            pltpu.VMEM((_CKS,), jnp.int32),
            pltpu.VMEM((_CKS,), jnp.int32),
            pltpu.SemaphoreType.DMA,
            pltpu.SemaphoreType.DMA,
        ],
    )
    def scat_kernel(rows_hbm, cols_hbm, g_hbm, zeros_hbm, out_hbm,
                    acc, ridx, b0, b1, a0, a1, c0, c1, semg, sems):
        bufs = (b0, b1)
        aidxs = (a0, a1)
        cidxs = (c0, c1)
        c = lax.axis_index("c")
        s = lax.axis_index("s")
        roff = c * n
        pltpu.sync_copy(zeros_hbm.at[pl.ds(s * rpt, rpt)],
                        acc.at[pl.ds(s * rpt, rpt)])
        plsc.subcore_barrier()
        base = s * ept

        def load_fire(i, sl):
            off = base + i * _CKS
            pltpu.sync_copy(rows_hbm.at[pl.ds(off, _CKS)], ridx)
            pltpu.sync_copy(cols_hbm.at[pl.ds(off, _CKS)], cidxs[sl])
            for gi in range(_CKS // _LN):
                lanes = pl.ds(gi * _LN, _LN)
                aidxs[sl][lanes] = ridx[lanes] + roff
            pltpu.async_copy(g_hbm.at[aidxs[sl]], bufs[sl], semg)

        def wait_gather(sl):
            pltpu.make_async_copy(g_hbm.at[aidxs[sl]], bufs[sl], semg).wait()

        def wait_scat(sl):
            pltpu.make_async_copy(zeros_hbm.at[pl.ds(0, _CKS)],
                                  bufs[sl], sems).wait()

        load_fire(0, 0)
        pltpu.sync_copy(zeros_hbm.at[pl.ds(0, _CKS)], bufs[1])
        pltpu.sync_copy(cols_hbm.at[pl.ds(base, _CKS)], cidxs[1])
        pltpu.async_copy(bufs[1], acc.at[cidxs[1]], sems, add=True)

        def step(i, cur, nxt):
            wait_gather(cur)
            wait_scat(nxt)
            load_fire(i + 1, nxt)
            pltpu.async_copy(bufs[cur], acc.at[cidxs[cur]], sems, add=True)

        def body(j, carry):
            step(2 * j, 0, 1)
            step(2 * j + 1, 1, 0)
            return carry

        lax.fori_loop(0, npairs, body, 0)
        wait_gather(0)
        wait_scat(1)
        pltpu.sync_copy(bufs[0], acc.at[cidxs[0]], add=True)
        plsc.subcore_barrier()
        pltpu.sync_copy(acc.at[pl.ds(s * rpt, rpt)],
                        out_hbm.at[c, pl.ds(s * rpt, rpt)])

    return scat_kernel


def _dot(a, b):
    return jnp.dot(a, b, precision=lax.Precision.HIGHEST,
                   preferred_element_type=jnp.float32)


def _split2(g):
    half = g.shape[1] // 2
    return jnp.concatenate([g[None, :, :half], g[None, :, half:]], axis=0)


def _tc_a1_body(x_ref, w_ref, h_ref):
    h_ref[...] = _dot(x_ref[...], w_ref[...])


@functools.lru_cache(maxsize=None)
def _tc_a1(n, d, h, blk):
    return pl.pallas_call(
        _tc_a1_body,
        grid=(n // blk,),
        in_specs=[
            pl.BlockSpec((blk, d), lambda b: (b, 0)),
            pl.BlockSpec((d, h), lambda b: (0, 0)),
        ],
        out_specs=pl.BlockSpec((blk, h), lambda b: (b, 0)),
        out_shape=jax.ShapeDtypeStruct((n, h), jnp.float32),
    )


def _tc_a2_body(deg_ref, h_ref, g_ref, dinv_ref):
    deg = deg_ref[0, :, :_LN] + deg_ref[1, :, :_LN] + 1.0
    dinv = lax.rsqrt(deg)
    g = h_ref[...] * dinv[:, 0:1]
    g_ref[...] = _split2(g)
    dinv_ref[...] = dinv


@functools.lru_cache(maxsize=None)
def _tc_a2(n, h, blk):
    return pl.pallas_call(
        _tc_a2_body,
        grid=(n // blk,),
        in_specs=[
            pl.BlockSpec((_NC, blk, h // 2), lambda b: (0, b, 0)),
            pl.BlockSpec((blk, h), lambda b: (b, 0)),
        ],
        out_specs=[
            pl.BlockSpec((2, blk, h // 2), lambda b: (0, b, 0)),
            pl.BlockSpec((blk, _LN), lambda b: (b, 0)),
        ],
        out_shape=[
            jax.ShapeDtypeStruct((2, n, h // 2), jnp.float32),
            jax.ShapeDtypeStruct((n, _LN), jnp.float32),
        ],
    )


def _tc_b_body(alo_ref, ahi_ref, h1_ref, dinv_ref, b1_ref, w2_ref,
               h2_ref, g_ref):
    d1 = dinv_ref[:, 0:1]
    acc = jnp.concatenate([alo_ref[...], ahi_ref[...]], axis=1)
    out1 = d1 * acc + (d1 * d1) * h1_ref[...] + b1_ref[...]
    m = jnp.maximum(out1, 0.0)
    h2 = _dot(m, w2_ref[...])
    g2 = h2 * d1
    h2_ref[...] = h2
    g_ref[...] = _split2(g2)


@functools.lru_cache(maxsize=None)
def _tc_b(n, h, blk):
    return pl.pallas_call(
        _tc_b_body,
        grid=(n // blk,),
        in_specs=[
            pl.BlockSpec((blk, h // 2), lambda b: (b, 0)),
            pl.BlockSpec((blk, h // 2), lambda b: (b, 0)),
            pl.BlockSpec((blk, h), lambda b: (b, 0)),
            pl.BlockSpec((blk, _LN), lambda b: (b, 0)),
            pl.BlockSpec((1, h), lambda b: (0, 0)),
            pl.BlockSpec((h, h), lambda b: (0, 0)),
        ],
        out_specs=[
            pl.BlockSpec((blk, h), lambda b: (b, 0)),
            pl.BlockSpec((2, blk, h // 2), lambda b: (0, b, 0)),
        ],
        out_shape=[
            jax.ShapeDtypeStruct((n, h), jnp.float32),
            jax.ShapeDtypeStruct((2, n, h // 2), jnp.float32),
        ],
    )


def _tc_c_body(alo_ref, ahi_ref, h2_ref, dinv_ref, b2_ref, wo_ref, bo_ref,
               out_ref):
    d1 = dinv_ref[:, 0:1]
    acc = jnp.concatenate([alo_ref[...], ahi_ref[...]], axis=1)
    out2 = d1 * acc + (d1 * d1) * h2_ref[...] + b2_ref[...]
    logits = _dot(out2, wo_ref[...]) + bo_ref[...]
    mx = jnp.max(logits, axis=1, keepdims=True)
    sh = logits - mx
    lse = jnp.log(jnp.sum(jnp.exp(sh), axis=1, keepdims=True))
    out_ref[...] = sh - lse


@functools.lru_cache(maxsize=None)
def _tc_c(n, h, cdim, blk):
    return pl.pallas_call(
        _tc_c_body,
        grid=(n // blk,),
        in_specs=[
            pl.BlockSpec((blk, h // 2), lambda b: (b, 0)),
            pl.BlockSpec((blk, h // 2), lambda b: (b, 0)),
            pl.BlockSpec((blk, h), lambda b: (b, 0)),
            pl.BlockSpec((blk, _LN), lambda b: (b, 0)),
            pl.BlockSpec((1, h), lambda b: (0, 0)),
            pl.BlockSpec((h, cdim), lambda b: (0, 0)),
            pl.BlockSpec((1, cdim), lambda b: (0, 0)),
        ],
        out_specs=pl.BlockSpec((blk, cdim), lambda b: (b, 0)),
        out_shape=jax.ShapeDtypeStruct((n, cdim), jnp.float32),
    )


def kernel(x, edge_index, W1, b1, W2, b2, Wo, bo):
    n, d = x.shape
    e = edge_index.shape[1]
    h = W1.shape[1]
    cdim = Wo.shape[1]
    f = h // 2
    blk = 400

    npad = _pad_n(n)
    rows = edge_index[0]
    cols = edge_index[1]
    ones_d = jnp.ones((_CKD, f), jnp.float32)
    zeros_f = jnp.zeros((npad, f), jnp.float32)

    h1 = _tc_a1(n, d, h, blk)(x, W1)
    degs = _degree_sc(n, e, f)(cols, zeros_f, ones_d)[:, :n]
    g1, dinv = _tc_a2(n, h, blk)(degs, h1)
    acc1 = _scatter_sc(n, e, f)(rows, cols, g1.reshape(2 * n, f), zeros_f)
    h2, g2 = _tc_b(n, h, blk)(
        acc1[0, :n], acc1[1, :n], h1, dinv, b1.reshape(1, -1), W2)
    acc2 = _scatter_sc(n, e, f)(rows, cols, g2.reshape(2 * n, f), zeros_f)
    return _tc_c(n, h, cdim, blk)(
        acc2[0, :n], acc2[1, :n], h2, dinv, b2.reshape(1, -1), Wo, bo.reshape(1, -1))


# speedup vs baseline: 9.7306x; 1.2220x over previous
import functools

import jax
import jax.numpy as jnp
from jax import lax
from jax.experimental import pallas as pl
from jax.experimental.pallas import tpu as pltpu
from jax.experimental.pallas import tpu_sc as plsc

_NC = 2
_NS = 16
_LN = 16
_CKD = 40
_CKS = 80


def _pad_n(n):
    step = _NS * 8
    return ((n + step - 1) // step) * step


@functools.lru_cache(maxsize=None)
def _degree_sc(n, e, fw):
    epc = e // _NC
    ept = epc // _NS
    nchunks = ept // _CKD
    npad = _pad_n(n)
    rpt = npad // _NS

    mesh = plsc.VectorSubcoreMesh(
        core_axis_name="c", subcore_axis_name="s",
        num_cores=_NC, num_subcores=_NS)

    @functools.partial(
        pl.kernel,
        out_type=jax.ShapeDtypeStruct((_NC, npad, fw), jnp.float32),
        mesh=mesh,
        scratch_types=[
            pltpu.VMEM_SHARED((npad, fw), jnp.float32),
            pltpu.VMEM((_CKD, fw), jnp.float32),
            pltpu.VMEM((nchunks, _CKD), jnp.int32),
        ],
    )
    def deg_kernel(colsd_hbm, zeros_hbm, ones_hbm, out_hbm, acc, ones_v, colpre):
        c = lax.axis_index("c")
        s = lax.axis_index("s")
        pltpu.sync_copy(zeros_hbm.at[pl.ds(s * rpt, rpt)],
                        acc.at[pl.ds(s * rpt, rpt)])
        pltpu.sync_copy(ones_hbm, ones_v)
        pltpu.sync_copy(colsd_hbm.at[c, s], colpre)
        plsc.subcore_barrier()

        def body(i, carry):
            pltpu.sync_copy(ones_v, acc.at[colpre.at[i]], add=True)
            return carry

        lax.fori_loop(0, nchunks, body, 0)
        plsc.subcore_barrier()
        pltpu.sync_copy(acc.at[pl.ds(s * rpt, rpt)],
                        out_hbm.at[c, pl.ds(s * rpt, rpt)])

    return deg_kernel


@functools.lru_cache(maxsize=None)
def _scatter_sc(n, e, f):
    ept = e // _NS
    nchunks = ept // _CKS
    npairs = (nchunks - 1) // 2
    npad = _pad_n(n)
    rpt = npad // _NS

    mesh = plsc.VectorSubcoreMesh(
        core_axis_name="c", subcore_axis_name="s",
        num_cores=_NC, num_subcores=_NS)

    @functools.partial(
        pl.kernel,
        out_type=jax.ShapeDtypeStruct((_NC, npad, f), jnp.float32),
        mesh=mesh,
        scratch_types=[
            pltpu.VMEM_SHARED((npad, f), jnp.float32),
            pltpu.VMEM((_CKS, f), jnp.float32),
            pltpu.VMEM((_CKS, f), jnp.float32),
            pltpu.VMEM((2, _CKS), jnp.int32),
            pltpu.VMEM((2, _CKS), jnp.int32),
            pltpu.SemaphoreType.DMA,
            pltpu.SemaphoreType.DMA,
        ],
    )
    def scat_kernel(combo_hbm, g_hbm, zeros_hbm, out_hbm,
                    acc, b0, b1, i0, i1, semg, sems):
        bufs = (b0, b1)
        idx2s = (i0, i1)
        c = lax.axis_index("c")
        s = lax.axis_index("s")
        pltpu.sync_copy(zeros_hbm.at[pl.ds(s * rpt, rpt)],
                        acc.at[pl.ds(s * rpt, rpt)])
        plsc.subcore_barrier()

        def load_fire(i, sl):
            pltpu.sync_copy(combo_hbm.at[c, s, i], idx2s[sl])
            pltpu.async_copy(g_hbm.at[idx2s[sl].at[0]], bufs[sl], semg)

        def wait_gather(sl):
            pltpu.make_async_copy(g_hbm.at[idx2s[sl].at[0]], bufs[sl],
                                  semg).wait()

        def wait_scat(sl):
            pltpu.make_async_copy(zeros_hbm.at[pl.ds(0, _CKS)],
                                  bufs[sl], sems).wait()

        load_fire(0, 0)
        pltpu.sync_copy(zeros_hbm.at[pl.ds(0, _CKS)], bufs[1])
        pltpu.sync_copy(combo_hbm.at[c, s, 0], idx2s[1])
        pltpu.async_copy(bufs[1], acc.at[idx2s[1].at[1]], sems, add=True)

        def step(i, cur, nxt):
            wait_gather(cur)
            wait_scat(nxt)
            load_fire(i + 1, nxt)
            pltpu.async_copy(bufs[cur], acc.at[idx2s[cur].at[1]], sems,
                             add=True)

        def body(j, carry):
            step(2 * j, 0, 1)
            step(2 * j + 1, 1, 0)
            return carry

        lax.fori_loop(0, npairs, body, 0)
        wait_gather(0)
        wait_scat(1)
        pltpu.sync_copy(bufs[0], acc.at[idx2s[0].at[1]], add=True)
        plsc.subcore_barrier()
        pltpu.sync_copy(acc.at[pl.ds(s * rpt, rpt)],
                        out_hbm.at[c, pl.ds(s * rpt, rpt)])

    return scat_kernel


def _dot(a, b):
    return jnp.dot(a, b, precision=lax.Precision.HIGHEST,
                   preferred_element_type=jnp.float32)


def _split2(g):
    half = g.shape[1] // 2
    return jnp.concatenate([g[None, :, :half], g[None, :, half:]], axis=0)


def _tc_a1_body(x_ref, w_ref, h_ref):
    h_ref[...] = _dot(x_ref[...], w_ref[...])


@functools.lru_cache(maxsize=None)
def _tc_a1(n, d, h, blk):
    return pl.pallas_call(
        _tc_a1_body,
        grid=(n // blk,),
        in_specs=[
            pl.BlockSpec((blk, d), lambda b: (b, 0)),
            pl.BlockSpec((d, h), lambda b: (0, 0)),
        ],
        out_specs=pl.BlockSpec((blk, h), lambda b: (b, 0)),
        out_shape=jax.ShapeDtypeStruct((n, h), jnp.float32),
    )


def _tc_a2_body(deg_ref, h_ref, g_ref, dinv_ref):
    deg = deg_ref[0, :, :_LN] + deg_ref[1, :, :_LN] + 1.0
    dinv = lax.rsqrt(deg)
    g = h_ref[...] * dinv[:, 0:1]
    g_ref[...] = _split2(g)
    dinv_ref[...] = dinv


@functools.lru_cache(maxsize=None)
def _tc_a2(n, h, blk):
    return pl.pallas_call(
        _tc_a2_body,
        grid=(n // blk,),
        in_specs=[
            pl.BlockSpec((_NC, blk, h // 2), lambda b: (0, b, 0)),
            pl.BlockSpec((blk, h), lambda b: (b, 0)),
        ],
        out_specs=[
            pl.BlockSpec((2, blk, h // 2), lambda b: (0, b, 0)),
            pl.BlockSpec((blk, _LN), lambda b: (b, 0)),
        ],
        out_shape=[
            jax.ShapeDtypeStruct((2, n, h // 2), jnp.float32),
            jax.ShapeDtypeStruct((n, _LN), jnp.float32),
        ],
    )


def _tc_b_body(alo_ref, ahi_ref, h1_ref, dinv_ref, b1_ref, w2_ref,
               h2_ref, g_ref):
    d1 = dinv_ref[:, 0:1]
    acc = jnp.concatenate([alo_ref[...], ahi_ref[...]], axis=1)
    out1 = d1 * acc + (d1 * d1) * h1_ref[...] + b1_ref[...]
    m = jnp.maximum(out1, 0.0)
    h2 = _dot(m, w2_ref[...])
    g2 = h2 * d1
    h2_ref[...] = h2
    g_ref[...] = _split2(g2)


@functools.lru_cache(maxsize=None)
def _tc_b(n, h, blk):
    return pl.pallas_call(
        _tc_b_body,
        grid=(n // blk,),
        in_specs=[
            pl.BlockSpec((blk, h // 2), lambda b: (b, 0)),
            pl.BlockSpec((blk, h // 2), lambda b: (b, 0)),
            pl.BlockSpec((blk, h), lambda b: (b, 0)),
            pl.BlockSpec((blk, _LN), lambda b: (b, 0)),
            pl.BlockSpec((1, h), lambda b: (0, 0)),
            pl.BlockSpec((h, h), lambda b: (0, 0)),
        ],
        out_specs=[
            pl.BlockSpec((blk, h), lambda b: (b, 0)),
            pl.BlockSpec((2, blk, h // 2), lambda b: (0, b, 0)),
        ],
        out_shape=[
            jax.ShapeDtypeStruct((n, h), jnp.float32),
            jax.ShapeDtypeStruct((2, n, h // 2), jnp.float32),
        ],
    )


def _tc_c_body(alo_ref, ahi_ref, h2_ref, dinv_ref, b2_ref, wo_ref, bo_ref,
               out_ref):
    d1 = dinv_ref[:, 0:1]
    acc = jnp.concatenate([alo_ref[...], ahi_ref[...]], axis=1)
    out2 = d1 * acc + (d1 * d1) * h2_ref[...] + b2_ref[...]
    logits = _dot(out2, wo_ref[...]) + bo_ref[...]
    mx = jnp.max(logits, axis=1, keepdims=True)
    sh = logits - mx
    lse = jnp.log(jnp.sum(jnp.exp(sh), axis=1, keepdims=True))
    out_ref[...] = sh - lse


@functools.lru_cache(maxsize=None)
def _tc_c(n, h, cdim, blk):
    return pl.pallas_call(
        _tc_c_body,
        grid=(n // blk,),
        in_specs=[
            pl.BlockSpec((blk, h // 2), lambda b: (b, 0)),
            pl.BlockSpec((blk, h // 2), lambda b: (b, 0)),
            pl.BlockSpec((blk, h), lambda b: (b, 0)),
            pl.BlockSpec((blk, _LN), lambda b: (b, 0)),
            pl.BlockSpec((1, h), lambda b: (0, 0)),
            pl.BlockSpec((h, cdim), lambda b: (0, 0)),
            pl.BlockSpec((1, cdim), lambda b: (0, 0)),
        ],
        out_specs=pl.BlockSpec((blk, cdim), lambda b: (b, 0)),
        out_shape=jax.ShapeDtypeStruct((n, cdim), jnp.float32),
    )


def kernel(x, edge_index, W1, b1, W2, b2, Wo, bo):
    n, d = x.shape
    e = edge_index.shape[1]
    h = W1.shape[1]
    cdim = Wo.shape[1]
    f = h // 2
    blk = 400

    npad = _pad_n(n)
    rows = edge_index[0]
    cols = edge_index[1]
    ones_d = jnp.ones((_CKD, f), jnp.float32)
    zeros_f = jnp.zeros((npad, f), jnp.float32)

    nchunks = (e // _NS) // _CKS
    rows_r = rows.reshape(_NS, nchunks, _CKS)
    cols_r = cols.reshape(_NS, nchunks, _CKS)
    combo = jnp.stack([jnp.stack([rows_r, cols_r], axis=2),
                       jnp.stack([rows_r + n, cols_r], axis=2)])
    nchunks_d = (e // _NC // _NS) // _CKD
    cols_d = cols.reshape(_NC, _NS, nchunks_d, _CKD)

    h1 = _tc_a1(n, d, h, blk)(x, W1)
    degs = _degree_sc(n, e, f)(cols_d, zeros_f, ones_d)[:, :n]
    g1, dinv = _tc_a2(n, h, blk)(degs, h1)
    acc1 = _scatter_sc(n, e, f)(combo, g1.reshape(2 * n, f), zeros_f)
    h2, g2 = _tc_b(n, h, blk)(
        acc1[0, :n], acc1[1, :n], h1, dinv, b1.reshape(1, -1), W2)
    acc2 = _scatter_sc(n, e, f)(combo, g2.reshape(2 * n, f), zeros_f)
    return _tc_c(n, h, cdim, blk)(
        acc2[0, :n], acc2[1, :n], h2, dinv, b2.reshape(1, -1), Wo, bo.reshape(1, -1))


# speedup vs baseline: 11.9146x; 1.2245x over previous
import functools

import jax
import jax.numpy as jnp
from jax import lax
from jax.experimental import pallas as pl
from jax.experimental.pallas import tpu as pltpu
from jax.experimental.pallas import tpu_sc as plsc

_NC = 2
_NS = 16
_LN = 16
_CKD = 40
_CKS = 80
_NBLK = 5


def _pad_n(n):
    step = _NS * 8
    return ((n + step - 1) // step) * step


@functools.lru_cache(maxsize=None)
def _degree_sc(n, e, fw):
    epc = e // _NC
    ept = epc // _NS
    nchunks = ept // _CKD
    npad = _pad_n(n)
    rpt = npad // _NS

    mesh = plsc.VectorSubcoreMesh(
        core_axis_name="c", subcore_axis_name="s",
        num_cores=_NC, num_subcores=_NS)

    @functools.partial(
        pl.kernel,
        out_type=jax.ShapeDtypeStruct((_NC, npad, fw), jnp.float32),
        mesh=mesh,
        scratch_types=[
            pltpu.VMEM_SHARED((npad, fw), jnp.float32),
            pltpu.VMEM((_CKD, fw), jnp.float32),
            pltpu.VMEM((nchunks, _CKD), jnp.int32),
        ],
    )
    def deg_kernel(colsd_hbm, zeros_hbm, ones_hbm, out_hbm, acc, ones_v, colpre):
        c = lax.axis_index("c")
        s = lax.axis_index("s")
        pltpu.sync_copy(zeros_hbm.at[pl.ds(s * rpt, rpt)],
                        acc.at[pl.ds(s * rpt, rpt)])
        pltpu.sync_copy(ones_hbm, ones_v)
        pltpu.sync_copy(colsd_hbm.at[c, s], colpre)
        plsc.subcore_barrier()

        def body(i, carry):
            pltpu.sync_copy(ones_v, acc.at[colpre.at[i]], add=True)
            return carry

        lax.fori_loop(0, nchunks, body, 0)
        plsc.subcore_barrier()
        pltpu.sync_copy(acc.at[pl.ds(s * rpt, rpt)],
                        out_hbm.at[c, pl.ds(s * rpt, rpt)])

    return deg_kernel


@functools.lru_cache(maxsize=None)
def _scatter_sc(n, e, f):
    ept = e // _NS
    nchunks = ept // _CKS
    npad = _pad_n(n)
    rpt = npad // _NS
    nblk = _NBLK
    kpb = nchunks // nblk
    kpairs = (kpb - 1) // 2

    mesh = plsc.VectorSubcoreMesh(
        core_axis_name="c", subcore_axis_name="s",
        num_cores=_NC, num_subcores=_NS)

    @functools.partial(
        pl.kernel,
        out_type=jax.ShapeDtypeStruct((_NC, npad, f), jnp.float32),
        mesh=mesh,
        scratch_types=[
            pltpu.VMEM_SHARED((npad, f), jnp.float32),
            pltpu.VMEM((_CKS, f), jnp.float32),
            pltpu.VMEM((_CKS, f), jnp.float32),
            pltpu.VMEM((2 * kpb, _CKS), jnp.int32),
            pltpu.VMEM((2 * kpb, _CKS), jnp.int32),
            pltpu.SemaphoreType.DMA,
            pltpu.SemaphoreType.DMA,
        ],
    )
    def scat_kernel(combo_hbm, g_hbm, zeros_hbm, out_hbm,
                    acc, b0, b1, i0, i1, semg, sems):
        bufs = (b0, b1)
        iblks = (i0, i1)
        c = lax.axis_index("c")
        s = lax.axis_index("s")
        pltpu.sync_copy(zeros_hbm.at[pl.ds(s * rpt, rpt)],
                        acc.at[pl.ds(s * rpt, rpt)])
        plsc.subcore_barrier()

        def wait_gather(dsl):
            pltpu.make_async_copy(g_hbm.at[iblks[0].at[0]], bufs[dsl],
                                  semg).wait()

        def wait_scat(dsl):
            pltpu.make_async_copy(zeros_hbm.at[pl.ds(0, _CKS)],
                                  bufs[dsl], sems).wait()

        def fire_gather(isl, q, dsl):
            pltpu.async_copy(g_hbm.at[iblks[isl].at[2 * q]], bufs[dsl], semg)

        def fire_scat(isl, q, dsl):
            pltpu.async_copy(bufs[dsl], acc.at[iblks[isl].at[2 * q + 1]],
                             sems, add=True)

        pltpu.sync_copy(combo_hbm.at[c, s, 0], iblks[0])
        fire_gather(0, 0, 0)
        pltpu.sync_copy(zeros_hbm.at[pl.ds(0, _CKS)], bufs[1])
        pltpu.async_copy(bufs[1], acc.at[iblks[0].at[1]], sems, add=True)

        for bb in range(nblk):
            isl = bb % 2

            def pair(j, carry, _isl=isl, _bb=bb):
                for k in range(2):
                    q = 2 * j + k
                    dsl = (_bb + k) % 2
                    wait_gather(dsl)
                    wait_scat(1 - dsl)
                    fire_gather(_isl, q + 1, 1 - dsl)
                    fire_scat(_isl, q, dsl)
                return carry

            lax.fori_loop(0, kpairs, pair, 0)
            dsl = bb % 2
            wait_gather(dsl)
            wait_scat(1 - dsl)
            if bb + 1 < nblk:
                pltpu.sync_copy(combo_hbm.at[c, s, bb + 1],
                                iblks[(bb + 1) % 2])
                fire_gather((bb + 1) % 2, 0, 1 - dsl)
                fire_scat(isl, kpb - 1, dsl)
            else:
                pltpu.sync_copy(bufs[dsl],
                                acc.at[iblks[isl].at[2 * (kpb - 1) + 1]],
                                add=True)
        plsc.subcore_barrier()
        pltpu.sync_copy(acc.at[pl.ds(s * rpt, rpt)],
                        out_hbm.at[c, pl.ds(s * rpt, rpt)])

    return scat_kernel


def _dot(a, b):
    return jnp.dot(a, b, precision=lax.Precision.HIGHEST,
                   preferred_element_type=jnp.float32)


def _split2(g):
    half = g.shape[1] // 2
    return jnp.concatenate([g[None, :, :half], g[None, :, half:]], axis=0)


def _tc_a1_body(x_ref, w_ref, h_ref):
    h_ref[...] = _dot(x_ref[...], w_ref[...])


@functools.lru_cache(maxsize=None)
def _tc_a1(n, d, h, blk):
    return pl.pallas_call(
        _tc_a1_body,
        grid=(n // blk,),
        in_specs=[
            pl.BlockSpec((blk, d), lambda b: (b, 0)),
            pl.BlockSpec((d, h), lambda b: (0, 0)),
        ],
        out_specs=pl.BlockSpec((blk, h), lambda b: (b, 0)),
        out_shape=jax.ShapeDtypeStruct((n, h), jnp.float32),
    )


def _tc_a2_body(deg_ref, h_ref, g_ref, dinv_ref):
    deg = deg_ref[0, :, :_LN] + deg_ref[1, :, :_LN] + 1.0
    dinv = lax.rsqrt(deg)
    g = h_ref[...] * dinv[:, 0:1]
    g_ref[...] = _split2(g)
    dinv_ref[...] = dinv


@functools.lru_cache(maxsize=None)
def _tc_a2(n, h, blk):
    return pl.pallas_call(
        _tc_a2_body,
        grid=(n // blk,),
        in_specs=[
            pl.BlockSpec((_NC, blk, h // 2), lambda b: (0, b, 0)),
            pl.BlockSpec((blk, h), lambda b: (b, 0)),
        ],
        out_specs=[
            pl.BlockSpec((2, blk, h // 2), lambda b: (0, b, 0)),
            pl.BlockSpec((blk, _LN), lambda b: (b, 0)),
        ],
        out_shape=[
            jax.ShapeDtypeStruct((2, n, h // 2), jnp.float32),
            jax.ShapeDtypeStruct((n, _LN), jnp.float32),
        ],
    )


def _tc_b_body(alo_ref, ahi_ref, h1_ref, dinv_ref, b1_ref, w2_ref,
               h2_ref, g_ref):
    d1 = dinv_ref[:, 0:1]
    acc = jnp.concatenate([alo_ref[...], ahi_ref[...]], axis=1)
    out1 = d1 * acc + (d1 * d1) * h1_ref[...] + b1_ref[...]
    m = jnp.maximum(out1, 0.0)
    h2 = _dot(m, w2_ref[...])
    g2 = h2 * d1
    h2_ref[...] = h2
    g_ref[...] = _split2(g2)


@functools.lru_cache(maxsize=None)
def _tc_b(n, h, blk):
    return pl.pallas_call(
        _tc_b_body,
        grid=(n // blk,),
        in_specs=[
            pl.BlockSpec((blk, h // 2), lambda b: (b, 0)),
            pl.BlockSpec((blk, h // 2), lambda b: (b, 0)),
            pl.BlockSpec((blk, h), lambda b: (b, 0)),
            pl.BlockSpec((blk, _LN), lambda b: (b, 0)),
            pl.BlockSpec((1, h), lambda b: (0, 0)),
            pl.BlockSpec((h, h), lambda b: (0, 0)),
        ],
        out_specs=[
            pl.BlockSpec((blk, h), lambda b: (b, 0)),
            pl.BlockSpec((2, blk, h // 2), lambda b: (0, b, 0)),
        ],
        out_shape=[
            jax.ShapeDtypeStruct((n, h), jnp.float32),
            jax.ShapeDtypeStruct((2, n, h // 2), jnp.float32),
        ],
    )


def _tc_c_body(alo_ref, ahi_ref, h2_ref, dinv_ref, b2_ref, wo_ref, bo_ref,
               out_ref):
    d1 = dinv_ref[:, 0:1]
    acc = jnp.concatenate([alo_ref[...], ahi_ref[...]], axis=1)
    out2 = d1 * acc + (d1 * d1) * h2_ref[...] + b2_ref[...]
    logits = _dot(out2, wo_ref[...]) + bo_ref[...]
    mx = jnp.max(logits, axis=1, keepdims=True)
    sh = logits - mx
    lse = jnp.log(jnp.sum(jnp.exp(sh), axis=1, keepdims=True))
    out_ref[...] = sh - lse


@functools.lru_cache(maxsize=None)
def _tc_c(n, h, cdim, blk):
    return pl.pallas_call(
        _tc_c_body,
        grid=(n // blk,),
        in_specs=[
            pl.BlockSpec((blk, h // 2), lambda b: (b, 0)),
            pl.BlockSpec((blk, h // 2), lambda b: (b, 0)),
            pl.BlockSpec((blk, h), lambda b: (b, 0)),
            pl.BlockSpec((blk, _LN), lambda b: (b, 0)),
            pl.BlockSpec((1, h), lambda b: (0, 0)),
            pl.BlockSpec((h, cdim), lambda b: (0, 0)),
            pl.BlockSpec((1, cdim), lambda b: (0, 0)),
        ],
        out_specs=pl.BlockSpec((blk, cdim), lambda b: (b, 0)),
        out_shape=jax.ShapeDtypeStruct((n, cdim), jnp.float32),
    )


def kernel(x, edge_index, W1, b1, W2, b2, Wo, bo):
    n, d = x.shape
    e = edge_index.shape[1]
    h = W1.shape[1]
    cdim = Wo.shape[1]
    f = h // 2
    blk = 400

    npad = _pad_n(n)
    rows = edge_index[0]
    cols = edge_index[1]
    ones_d = jnp.ones((_CKD, f), jnp.float32)
    zeros_f = jnp.zeros((npad, f), jnp.float32)

    nchunks = (e // _NS) // _CKS
    rows_r = rows.reshape(_NS, nchunks, _CKS)
    cols_r = cols.reshape(_NS, nchunks, _CKS)
    combo = jnp.stack([jnp.stack([rows_r, cols_r], axis=2),
                       jnp.stack([rows_r + n, cols_r], axis=2)])
    combo = combo.reshape(2, _NS, _NBLK, 2 * (nchunks // _NBLK), _CKS)
    nchunks_d = (e // _NC // _NS) // _CKD
    cols_d = cols.reshape(_NC, _NS, nchunks_d, _CKD)

    h1 = _tc_a1(n, d, h, blk)(x, W1)
    degs = _degree_sc(n, e, f)(cols_d, zeros_f, ones_d)[:, :n]
    g1, dinv = _tc_a2(n, h, blk)(degs, h1)
    acc1 = _scatter_sc(n, e, f)(combo, g1.reshape(2 * n, f), zeros_f)
    h2, g2 = _tc_b(n, h, blk)(
        acc1[0, :n], acc1[1, :n], h1, dinv, b1.reshape(1, -1), W2)
    acc2 = _scatter_sc(n, e, f)(combo, g2.reshape(2 * n, f), zeros_f)
    return _tc_c(n, h, cdim, blk)(
        acc2[0, :n], acc2[1, :n], h2, dinv, b2.reshape(1, -1), Wo, bo.reshape(1, -1))


# speedup vs baseline: 12.5795x; 1.0558x over previous
import functools

import jax
import jax.numpy as jnp
from jax import lax
from jax.experimental import pallas as pl
from jax.experimental.pallas import tpu as pltpu
from jax.experimental.pallas import tpu_sc as plsc

_NC = 2
_NS = 16
_LN = 16
_CKD = 40
_CKS = 80
_NBLK = 5


def _pad_n(n):
    step = _NS * 8
    return ((n + step - 1) // step) * step


@functools.lru_cache(maxsize=None)
def _degree_sc(n, e, fw):
    epc = e // _NC
    ept = epc // _NS
    nchunks = ept // _CKD
    npad = _pad_n(n)
    rpt = npad // _NS

    mesh = plsc.VectorSubcoreMesh(
        core_axis_name="c", subcore_axis_name="s",
        num_cores=_NC, num_subcores=_NS)

    @functools.partial(
        pl.kernel,
        out_type=jax.ShapeDtypeStruct((_NC, npad, fw), jnp.float32),
        mesh=mesh,
        scratch_types=[
            pltpu.VMEM_SHARED((npad, fw), jnp.float32),
            pltpu.VMEM((_CKD, fw), jnp.float32),
            pltpu.VMEM((nchunks, _CKD), jnp.int32),
        ],
    )
    def deg_kernel(colsd_hbm, zeros_hbm, ones_hbm, out_hbm, acc, ones_v, colpre):
        c = lax.axis_index("c")
        s = lax.axis_index("s")
        pltpu.sync_copy(zeros_hbm.at[pl.ds(s * rpt, rpt)],
                        acc.at[pl.ds(s * rpt, rpt)])
        pltpu.sync_copy(ones_hbm, ones_v)
        pltpu.sync_copy(colsd_hbm.at[c, s], colpre)
        plsc.subcore_barrier()

        def body(i, carry):
            pltpu.sync_copy(ones_v, acc.at[colpre.at[i]], add=True)
            return carry

        lax.fori_loop(0, nchunks, body, 0)
        plsc.subcore_barrier()
        pltpu.sync_copy(acc.at[pl.ds(s * rpt, rpt)],
                        out_hbm.at[c, pl.ds(s * rpt, rpt)])

    return deg_kernel


@functools.lru_cache(maxsize=None)
def _scatter_sc(n, e, f):
    ept = e // _NS
    nchunks = ept // _CKS
    npad = _pad_n(n)
    rpt = npad // _NS
    nblk = _NBLK
    kpb = nchunks // nblk
    kpairs = (kpb - 1) // 2

    mesh = plsc.VectorSubcoreMesh(
        core_axis_name="c", subcore_axis_name="s",
        num_cores=_NC, num_subcores=_NS)

    @functools.partial(
        pl.kernel,
        out_type=jax.ShapeDtypeStruct((_NC, npad, f), jnp.float32),
        mesh=mesh,
        scratch_types=[
            pltpu.VMEM_SHARED((npad, f), jnp.float32),
            pltpu.VMEM((_CKS, f), jnp.float32),
            pltpu.VMEM((_CKS, f), jnp.float32),
            pltpu.VMEM((2 * kpb, _CKS), jnp.int32),
            pltpu.VMEM((2 * kpb, _CKS), jnp.int32),
            pltpu.SemaphoreType.DMA,
            pltpu.SemaphoreType.DMA,
        ],
    )
    def scat_kernel(combo_hbm, g_hbm, zeros_hbm, out_hbm,
                    acc, b0, b1, i0, i1, semg, sems):
        bufs = (b0, b1)
        iblks = (i0, i1)
        c = lax.axis_index("c")
        s = lax.axis_index("s")
        pltpu.sync_copy(zeros_hbm.at[pl.ds(s * rpt, rpt)],
                        acc.at[pl.ds(s * rpt, rpt)])
        plsc.subcore_barrier()

        def wait_gather(dsl):
            pltpu.make_async_copy(g_hbm.at[iblks[0].at[0]], bufs[dsl],
                                  semg).wait()

        def wait_scat(dsl):
            pltpu.make_async_copy(zeros_hbm.at[pl.ds(0, _CKS)],
                                  bufs[dsl], sems).wait()

        def fire_gather(isl, q, dsl):
            pltpu.async_copy(g_hbm.at[iblks[isl].at[2 * q]], bufs[dsl], semg)

        def fire_scat(isl, q, dsl):
            pltpu.async_copy(bufs[dsl], acc.at[iblks[isl].at[2 * q + 1]],
                             sems, add=True)

        pltpu.sync_copy(combo_hbm.at[c, s, 0], iblks[0])
        fire_gather(0, 0, 0)
        pltpu.sync_copy(zeros_hbm.at[pl.ds(0, _CKS)], bufs[1])
        pltpu.async_copy(bufs[1], acc.at[iblks[0].at[1]], sems, add=True)

        for bb in range(nblk):
            isl = bb % 2

            def pair(j, carry, _isl=isl, _bb=bb):
                for k in range(2):
                    q = 2 * j + k
                    dsl = (_bb + k) % 2
                    wait_gather(dsl)
                    wait_scat(1 - dsl)
                    fire_gather(_isl, q + 1, 1 - dsl)
                    fire_scat(_isl, q, dsl)
                return carry

            lax.fori_loop(0, kpairs, pair, 0)
            dsl = bb % 2
            wait_gather(dsl)
            wait_scat(1 - dsl)
            if bb + 1 < nblk:
                pltpu.sync_copy(combo_hbm.at[c, s, bb + 1],
                                iblks[(bb + 1) % 2])
                fire_gather((bb + 1) % 2, 0, 1 - dsl)
                fire_scat(isl, kpb - 1, dsl)
            else:
                pltpu.sync_copy(bufs[dsl],
                                acc.at[iblks[isl].at[2 * (kpb - 1) + 1]],
                                add=True)
        plsc.subcore_barrier()
        pltpu.sync_copy(acc.at[pl.ds(s * rpt, rpt)],
                        out_hbm.at[c, pl.ds(s * rpt, rpt)])

    return scat_kernel


def _dot(a, b):
    return jnp.dot(a, b, precision=lax.Precision.HIGHEST,
                   preferred_element_type=jnp.float32)


def _split2(g):
    half = g.shape[1] // 2
    return jnp.concatenate([g[None, :, :half], g[None, :, half:]], axis=0)


def _tc_a1_body(x_ref, w_ref, h_ref):
    h_ref[...] = _dot(x_ref[...], w_ref[...])


@functools.lru_cache(maxsize=None)
def _tc_a1(n, d, h, blk):
    return pl.pallas_call(
        _tc_a1_body,
        grid=(n // blk,),
        in_specs=[
            pl.BlockSpec((blk, d), lambda b: (b, 0)),
            pl.BlockSpec((d, h), lambda b: (0, 0)),
        ],
        out_specs=pl.BlockSpec((blk, h), lambda b: (b, 0)),
        out_shape=jax.ShapeDtypeStruct((n, h), jnp.float32),
    )


def _tc_a2_body(deg_ref, h_ref, g_ref, dinv_ref):
    deg = deg_ref[0, :, :_LN] + deg_ref[1, :, :_LN] + 1.0
    dinv = lax.rsqrt(deg)
    g = h_ref[...] * dinv[:, 0:1]
    g_ref[...] = _split2(g)
    dinv_ref[...] = dinv


@functools.lru_cache(maxsize=None)
def _tc_a2(n, h, blk):
    return pl.pallas_call(
        _tc_a2_body,
        grid=(n // blk,),
        in_specs=[
            pl.BlockSpec((_NC, blk, h // 2), lambda b: (0, b, 0)),
            pl.BlockSpec((blk, h), lambda b: (b, 0)),
        ],
        out_specs=[
            pl.BlockSpec((2, blk, h // 2), lambda b: (0, b, 0)),
            pl.BlockSpec((blk, _LN), lambda b: (b, 0)),
        ],
        out_shape=[
            jax.ShapeDtypeStruct((2, n, h // 2), jnp.float32),
            jax.ShapeDtypeStruct((n, _LN), jnp.float32),
        ],
    )


def _tc_b_body(alo_ref, ahi_ref, h1_ref, dinv_ref, b1_ref, w2_ref,
               h2_ref, g_ref):
    d1 = dinv_ref[:, 0:1]
    acc = jnp.concatenate([alo_ref[...], ahi_ref[...]], axis=1)
    out1 = d1 * acc + (d1 * d1) * h1_ref[...] + b1_ref[...]
    m = jnp.maximum(out1, 0.0)
    h2 = _dot(m, w2_ref[...])
    g2 = h2 * d1
    h2_ref[...] = h2
    g_ref[...] = _split2(g2)


@functools.lru_cache(maxsize=None)
def _tc_b(n, h, blk):
    return pl.pallas_call(
        _tc_b_body,
        grid=(n // blk,),
        in_specs=[
            pl.BlockSpec((blk, h // 2), lambda b: (b, 0)),
            pl.BlockSpec((blk, h // 2), lambda b: (b, 0)),
            pl.BlockSpec((blk, h), lambda b: (b, 0)),
            pl.BlockSpec((blk, _LN), lambda b: (b, 0)),
            pl.BlockSpec((1, h), lambda b: (0, 0)),
            pl.BlockSpec((h, h), lambda b: (0, 0)),
        ],
        out_specs=[
            pl.BlockSpec((blk, h), lambda b: (b, 0)),
            pl.BlockSpec((2, blk, h // 2), lambda b: (0, b, 0)),
        ],
        out_shape=[
            jax.ShapeDtypeStruct((n, h), jnp.float32),
            jax.ShapeDtypeStruct((2, n, h // 2), jnp.float32),
        ],
    )


def _tc_c_body(alo_ref, ahi_ref, h2_ref, dinv_ref, b2_ref, wo_ref, bo_ref,
               out_ref):
    d1 = dinv_ref[:, 0:1]
    acc = jnp.concatenate([alo_ref[...], ahi_ref[...]], axis=1)
    out2 = d1 * acc + (d1 * d1) * h2_ref[...] + b2_ref[...]
    logits = _dot(out2, wo_ref[...]) + bo_ref[...]
    mx = jnp.max(logits, axis=1, keepdims=True)
    sh = logits - mx
    lse = jnp.log(jnp.sum(jnp.exp(sh), axis=1, keepdims=True))
    out_ref[...] = sh - lse


@functools.lru_cache(maxsize=None)
def _tc_c(n, h, cdim, blk):
    return pl.pallas_call(
        _tc_c_body,
        grid=(n // blk,),
        in_specs=[
            pl.BlockSpec((blk, h // 2), lambda b: (b, 0)),
            pl.BlockSpec((blk, h // 2), lambda b: (b, 0)),
            pl.BlockSpec((blk, h), lambda b: (b, 0)),
            pl.BlockSpec((blk, _LN), lambda b: (b, 0)),
            pl.BlockSpec((1, h), lambda b: (0, 0)),
            pl.BlockSpec((h, cdim), lambda b: (0, 0)),
            pl.BlockSpec((1, cdim), lambda b: (0, 0)),
        ],
        out_specs=pl.BlockSpec((blk, cdim), lambda b: (b, 0)),
        out_shape=jax.ShapeDtypeStruct((n, cdim), jnp.float32),
    )


def kernel(x, edge_index, W1, b1, W2, b2, Wo, bo):
    n, d = x.shape
    e = edge_index.shape[1]
    h = W1.shape[1]
    cdim = Wo.shape[1]
    f = h // 2
    blk = 1000

    npad = _pad_n(n)
    rows = edge_index[0]
    cols = edge_index[1]
    ones_d = jnp.ones((_CKD, f), jnp.float32)
    zeros_f = jnp.zeros((npad, f), jnp.float32)

    nchunks = (e // _NS) // _CKS
    rows_r = rows.reshape(_NS, nchunks, _CKS)
    cols_r = cols.reshape(_NS, nchunks, _CKS)
    combo = jnp.stack([jnp.stack([rows_r, cols_r], axis=2),
                       jnp.stack([rows_r + n, cols_r], axis=2)])
    combo = combo.reshape(2, _NS, _NBLK, 2 * (nchunks // _NBLK), _CKS)
    nchunks_d = (e // _NC // _NS) // _CKD
    cols_d = cols.reshape(_NC, _NS, nchunks_d, _CKD)

    h1 = _tc_a1(n, d, h, blk)(x, W1)
    degs = _degree_sc(n, e, f)(cols_d, zeros_f, ones_d)[:, :n]
    g1, dinv = _tc_a2(n, h, blk)(degs, h1)
    acc1 = _scatter_sc(n, e, f)(combo, g1.reshape(2 * n, f), zeros_f)
    h2, g2 = _tc_b(n, h, blk)(
        acc1[0, :n], acc1[1, :n], h1, dinv, b1.reshape(1, -1), W2)
    acc2 = _scatter_sc(n, e, f)(combo, g2.reshape(2 * n, f), zeros_f)
    return _tc_c(n, h, cdim, blk)(
        acc2[0, :n], acc2[1, :n], h2, dinv, b2.reshape(1, -1), Wo, bo.reshape(1, -1))


# speedup vs baseline: 12.6885x; 1.0087x over previous
import functools

import jax
import jax.numpy as jnp
from jax import lax
from jax.experimental import pallas as pl
from jax.experimental.pallas import tpu as pltpu
from jax.experimental.pallas import tpu_sc as plsc

_NC = 2
_NS = 16
_LN = 16
_CKD = 40
_CKS = 80
_NBLK = 5


def _pad_n(n):
    step = _NS * 8
    return ((n + step - 1) // step) * step


@functools.lru_cache(maxsize=None)
def _degree_sc(n, e, fw):
    epc = e // _NC
    ept = epc // _NS
    nchunks = ept // _CKD
    npad = _pad_n(n)
    rpt = npad // _NS

    mesh = plsc.VectorSubcoreMesh(
        core_axis_name="c", subcore_axis_name="s",
        num_cores=_NC, num_subcores=_NS)

    @functools.partial(
        pl.kernel,
        out_type=jax.ShapeDtypeStruct((_NC, npad, fw), jnp.float32),
        mesh=mesh,
        scratch_types=[
            pltpu.VMEM_SHARED((npad, fw), jnp.float32),
            pltpu.VMEM((_CKD, fw), jnp.float32),
            pltpu.VMEM((nchunks, _CKD), jnp.int32),
        ],
    )
    def deg_kernel(colsd_hbm, zeros_hbm, ones_hbm, out_hbm, acc, ones_v, colpre):
        c = lax.axis_index("c")
        s = lax.axis_index("s")
        pltpu.sync_copy(zeros_hbm.at[pl.ds(s * rpt, rpt)],
                        acc.at[pl.ds(s * rpt, rpt)])
        pltpu.sync_copy(ones_hbm, ones_v)
        pltpu.sync_copy(colsd_hbm.at[c, s], colpre)
        plsc.subcore_barrier()

        def body(i, carry):
            pltpu.sync_copy(ones_v, acc.at[colpre.at[i]], add=True)
            return carry

        lax.fori_loop(0, nchunks, body, 0)
        plsc.subcore_barrier()
        pltpu.sync_copy(acc.at[pl.ds(s * rpt, rpt)],
                        out_hbm.at[c, pl.ds(s * rpt, rpt)])

    return deg_kernel


@functools.lru_cache(maxsize=None)
def _scatter_sc(n, e, f):
    ept = e // _NS
    nchunks = ept // _CKS
    npad = _pad_n(n)
    rpt = npad // _NS
    nblk = _NBLK
    kpb = nchunks // nblk
    kpairs = (kpb - 1) // 2

    mesh = plsc.VectorSubcoreMesh(
        core_axis_name="c", subcore_axis_name="s",
        num_cores=_NC, num_subcores=_NS)

    @functools.partial(
        pl.kernel,
        out_type=jax.ShapeDtypeStruct((_NC, npad, f), jnp.float32),
        mesh=mesh,
        scratch_types=[
            pltpu.VMEM_SHARED((npad, f), jnp.float32),
            pltpu.VMEM((_CKS, f), jnp.float32),
            pltpu.VMEM((_CKS, f), jnp.float32),
            pltpu.VMEM((2 * kpb, _CKS), jnp.int32),
            pltpu.VMEM((2 * kpb, _CKS), jnp.int32),
            pltpu.SemaphoreType.DMA,
            pltpu.SemaphoreType.DMA,
        ],
    )
    def scat_kernel(combo_hbm, g_hbm, zeros_hbm, out_hbm,
                    acc, b0, b1, i0, i1, semg, sems):
        bufs = (b0, b1)
        iblks = (i0, i1)
        c = lax.axis_index("c")
        s = lax.axis_index("s")
        pltpu.sync_copy(zeros_hbm.at[pl.ds(s * rpt, rpt)],
                        acc.at[pl.ds(s * rpt, rpt)])
        plsc.subcore_barrier()

        def wait_gather(dsl):
            pltpu.make_async_copy(g_hbm.at[iblks[0].at[0]], bufs[dsl],
                                  semg).wait()

        def wait_scat(dsl):
            pltpu.make_async_copy(zeros_hbm.at[pl.ds(0, _CKS)],
                                  bufs[dsl], sems).wait()

        def fire_gather(isl, q, dsl):
            pltpu.async_copy(g_hbm.at[iblks[isl].at[2 * q]], bufs[dsl], semg)

        def fire_scat(isl, q, dsl):
            pltpu.async_copy(bufs[dsl], acc.at[iblks[isl].at[2 * q + 1]],
                             sems, add=True)

        pltpu.sync_copy(combo_hbm.at[c, s, 0], iblks[0])
        fire_gather(0, 0, 0)
        pltpu.sync_copy(zeros_hbm.at[pl.ds(0, _CKS)], bufs[1])
        pltpu.async_copy(bufs[1], acc.at[iblks[0].at[1]], sems, add=True)

        for bb in range(nblk):
            isl = bb % 2

            def pair(j, carry, _isl=isl, _bb=bb):
                for k in range(2):
                    q = 2 * j + k
                    dsl = (_bb + k) % 2
                    wait_gather(dsl)
                    wait_scat(1 - dsl)
                    fire_gather(_isl, q + 1, 1 - dsl)
                    fire_scat(_isl, q, dsl)
                return carry

            lax.fori_loop(0, kpairs, pair, 0)
            dsl = bb % 2
            wait_gather(dsl)
            wait_scat(1 - dsl)
            if bb + 1 < nblk:
                pltpu.sync_copy(combo_hbm.at[c, s, bb + 1],
                                iblks[(bb + 1) % 2])
                fire_gather((bb + 1) % 2, 0, 1 - dsl)
                fire_scat(isl, kpb - 1, dsl)
            else:
                pltpu.sync_copy(bufs[dsl],
                                acc.at[iblks[isl].at[2 * (kpb - 1) + 1]],
                                add=True)
        plsc.subcore_barrier()
        pltpu.sync_copy(acc.at[pl.ds(s * rpt, rpt)],
                        out_hbm.at[c, pl.ds(s * rpt, rpt)])

    return scat_kernel


def _dot(a, b):
    return jnp.dot(a, b, precision=lax.Precision.HIGHEST,
                   preferred_element_type=jnp.float32)


def _split2(g):
    half = g.shape[1] // 2
    return jnp.concatenate([g[None, :, :half], g[None, :, half:]], axis=0)


def _tc_a1_body(x_ref, w_ref, h_ref):
    h_ref[...] = _dot(x_ref[...], w_ref[...])


@functools.lru_cache(maxsize=None)
def _tc_a1(n, d, h, blk):
    return pl.pallas_call(
        _tc_a1_body,
        grid=(n // blk,),
        in_specs=[
            pl.BlockSpec((blk, d), lambda b: (b, 0)),
            pl.BlockSpec((d, h), lambda b: (0, 0)),
        ],
        out_specs=pl.BlockSpec((blk, h), lambda b: (b, 0)),
        out_shape=jax.ShapeDtypeStruct((n, h), jnp.float32),
    )


def _tc_a2_body(deg_ref, h_ref, g_ref, dinv_ref):
    deg = deg_ref[0, :, :_LN] + deg_ref[1, :, :_LN] + 1.0
    dinv = lax.rsqrt(deg)
    g = h_ref[...] * dinv[:, 0:1]
    g_ref[...] = _split2(g)
    dinv_ref[...] = dinv


@functools.lru_cache(maxsize=None)
def _tc_a2(n, h, blk):
    return pl.pallas_call(
        _tc_a2_body,
        grid=(n // blk,),
        in_specs=[
            pl.BlockSpec((_NC, blk, h // 2), lambda b: (0, b, 0)),
            pl.BlockSpec((blk, h), lambda b: (b, 0)),
        ],
        out_specs=[
            pl.BlockSpec((2, blk, h // 2), lambda b: (0, b, 0)),
            pl.BlockSpec((blk, _LN), lambda b: (b, 0)),
        ],
        out_shape=[
            jax.ShapeDtypeStruct((2, n, h // 2), jnp.float32),
            jax.ShapeDtypeStruct((n, _LN), jnp.float32),
        ],
    )


def _tc_b_body(alo_ref, ahi_ref, h1_ref, dinv_ref, b1_ref, w2_ref,
               h2_ref, g_ref):
    d1 = dinv_ref[:, 0:1]
    acc = jnp.concatenate([alo_ref[...], ahi_ref[...]], axis=1)
    out1 = d1 * acc + (d1 * d1) * h1_ref[...] + b1_ref[...]
    m = jnp.maximum(out1, 0.0)
    h2 = _dot(m, w2_ref[...])
    g2 = h2 * d1
    h2_ref[...] = h2
    g_ref[...] = _split2(g2)


@functools.lru_cache(maxsize=None)
def _tc_b(n, h, blk):
    return pl.pallas_call(
        _tc_b_body,
        grid=(n // blk,),
        in_specs=[
            pl.BlockSpec((blk, h // 2), lambda b: (b, 0)),
            pl.BlockSpec((blk, h // 2), lambda b: (b, 0)),
            pl.BlockSpec((blk, h), lambda b: (b, 0)),
            pl.BlockSpec((blk, _LN), lambda b: (b, 0)),
            pl.BlockSpec((1, h), lambda b: (0, 0)),
            pl.BlockSpec((h, h), lambda b: (0, 0)),
        ],
        out_specs=[
            pl.BlockSpec((blk, h), lambda b: (b, 0)),
            pl.BlockSpec((2, blk, h // 2), lambda b: (0, b, 0)),
        ],
        out_shape=[
            jax.ShapeDtypeStruct((n, h), jnp.float32),
            jax.ShapeDtypeStruct((2, n, h // 2), jnp.float32),
        ],
    )


def _tc_c_body(alo_ref, ahi_ref, h2_ref, dinv_ref, b2_ref, wo_ref, bo_ref,
               out_ref):
    d1 = dinv_ref[:, 0:1]
    acc = jnp.concatenate([alo_ref[...], ahi_ref[...]], axis=1)
    out2 = d1 * acc + (d1 * d1) * h2_ref[...] + b2_ref[...]
    logits = _dot(out2, wo_ref[...]) + bo_ref[...]
    mx = jnp.max(logits, axis=1, keepdims=True)
    sh = logits - mx
    lse = jnp.log(jnp.sum(jnp.exp(sh), axis=1, keepdims=True))
    out_ref[...] = sh - lse


@functools.lru_cache(maxsize=None)
def _tc_c(n, h, cdim, blk):
    return pl.pallas_call(
        _tc_c_body,
        grid=(n // blk,),
        in_specs=[
            pl.BlockSpec((blk, h // 2), lambda b: (b, 0)),
            pl.BlockSpec((blk, h // 2), lambda b: (b, 0)),
            pl.BlockSpec((blk, h), lambda b: (b, 0)),
            pl.BlockSpec((blk, _LN), lambda b: (b, 0)),
            pl.BlockSpec((1, h), lambda b: (0, 0)),
            pl.BlockSpec((h, cdim), lambda b: (0, 0)),
            pl.BlockSpec((1, cdim), lambda b: (0, 0)),
        ],
        out_specs=pl.BlockSpec((blk, cdim), lambda b: (b, 0)),
        out_shape=jax.ShapeDtypeStruct((n, cdim), jnp.float32),
    )


def kernel(x, edge_index, W1, b1, W2, b2, Wo, bo):
    n, d = x.shape
    e = edge_index.shape[1]
    h = W1.shape[1]
    cdim = Wo.shape[1]
    f = h // 2
    blk = 2000

    npad = _pad_n(n)
    rows = edge_index[0]
    cols = edge_index[1]
    ones_d = jnp.ones((_CKD, f), jnp.float32)
    zeros_f = jnp.zeros((npad, f), jnp.float32)

    nchunks = (e // _NS) // _CKS
    rows_r = rows.reshape(_NS, nchunks, _CKS)
    cols_r = cols.reshape(_NS, nchunks, _CKS)
    combo = jnp.stack([jnp.stack([rows_r, cols_r], axis=2),
                       jnp.stack([rows_r + n, cols_r], axis=2)])
    combo = combo.reshape(2, _NS, _NBLK, 2 * (nchunks // _NBLK), _CKS)
    nchunks_d = (e // _NC // _NS) // _CKD
    cols_d = cols.reshape(_NC, _NS, nchunks_d, _CKD)

    h1 = _tc_a1(n, d, h, blk)(x, W1)
    degs = _degree_sc(n, e, f)(cols_d, zeros_f, ones_d)[:, :n]
    g1, dinv = _tc_a2(n, h, blk)(degs, h1)
    acc1 = _scatter_sc(n, e, f)(combo, g1.reshape(2 * n, f), zeros_f)
    h2, g2 = _tc_b(n, h, blk)(
        acc1[0, :n], acc1[1, :n], h1, dinv, b1.reshape(1, -1), W2)
    acc2 = _scatter_sc(n, e, f)(combo, g2.reshape(2 * n, f), zeros_f)
    return _tc_c(n, h, cdim, blk)(
        acc2[0, :n], acc2[1, :n], h2, dinv, b2.reshape(1, -1), Wo, bo.reshape(1, -1))


# speedup vs baseline: 12.8037x; 1.0091x over previous
import functools

import jax
import jax.numpy as jnp
from jax import lax
from jax.experimental import pallas as pl
from jax.experimental.pallas import tpu as pltpu
from jax.experimental.pallas import tpu_sc as plsc

_NC = 2
_NS = 16
_LN = 16
_CKD = 40
_CKS = 80
_NBLK = 5


def _pad_n(n):
    step = _NS * 8
    return ((n + step - 1) // step) * step


@functools.lru_cache(maxsize=None)
def _degree_sc(n, e, fw):
    epc = e // _NC
    ept = epc // _NS
    nchunks = ept // _CKD
    npad = _pad_n(n)
    rpt = npad // _NS

    mesh = plsc.VectorSubcoreMesh(
        core_axis_name="c", subcore_axis_name="s",
        num_cores=_NC, num_subcores=_NS)

    @functools.partial(
        pl.kernel,
        out_type=jax.ShapeDtypeStruct((_NC, npad, fw), jnp.float32),
        mesh=mesh,
        scratch_types=[
            pltpu.VMEM_SHARED((npad, fw), jnp.float32),
            pltpu.VMEM((_CKD, fw), jnp.float32),
            pltpu.VMEM((nchunks, _CKD), jnp.int32),
            pltpu.SemaphoreType.DMA,
        ],
    )
    def deg_kernel(colsd_hbm, zeros_hbm, ones_hbm, out_hbm, acc, ones_v, colpre,
                   semd):
        c = lax.axis_index("c")
        s = lax.axis_index("s")
        pltpu.sync_copy(zeros_hbm.at[pl.ds(s * rpt, rpt)],
                        acc.at[pl.ds(s * rpt, rpt)])
        pltpu.sync_copy(ones_hbm, ones_v)
        pltpu.sync_copy(colsd_hbm.at[c, s], colpre)
        plsc.subcore_barrier()

        win = 2

        def fire(i):
            pltpu.async_copy(ones_v, acc.at[colpre.at[i]], semd, add=True)

        def drain():
            pltpu.make_async_copy(zeros_hbm.at[pl.ds(0, _CKD)], ones_v,
                                  semd).wait()

        for i in range(win):
            fire(i)

        def body(i, carry):
            drain()
            fire(i + win)
            return carry

        lax.fori_loop(0, nchunks - win, body, 0)
        for _ in range(win):
            drain()
        plsc.subcore_barrier()
        pltpu.sync_copy(acc.at[pl.ds(s * rpt, rpt)],
                        out_hbm.at[c, pl.ds(s * rpt, rpt)])

    return deg_kernel


@functools.lru_cache(maxsize=None)
def _scatter_sc(n, e, f):
    ept = e // _NS
    nchunks = ept // _CKS
    npad = _pad_n(n)
    rpt = npad // _NS
    nblk = _NBLK
    kpb = nchunks // nblk
    kpairs = (kpb - 1) // 2

    mesh = plsc.VectorSubcoreMesh(
        core_axis_name="c", subcore_axis_name="s",
        num_cores=_NC, num_subcores=_NS)

    @functools.partial(
        pl.kernel,
        out_type=jax.ShapeDtypeStruct((_NC, npad, f), jnp.float32),
        mesh=mesh,
        scratch_types=[
            pltpu.VMEM_SHARED((npad, f), jnp.float32),
            pltpu.VMEM((_CKS, f), jnp.float32),
            pltpu.VMEM((_CKS, f), jnp.float32),
            pltpu.VMEM((2 * kpb, _CKS), jnp.int32),
            pltpu.VMEM((2 * kpb, _CKS), jnp.int32),
            pltpu.SemaphoreType.DMA,
            pltpu.SemaphoreType.DMA,
        ],
    )
    def scat_kernel(combo_hbm, g_hbm, zeros_hbm, out_hbm,
                    acc, b0, b1, i0, i1, semg, sems):
        bufs = (b0, b1)
        iblks = (i0, i1)
        c = lax.axis_index("c")
        s = lax.axis_index("s")
        pltpu.sync_copy(zeros_hbm.at[pl.ds(s * rpt, rpt)],
                        acc.at[pl.ds(s * rpt, rpt)])
        plsc.subcore_barrier()

        def wait_gather(dsl):
            pltpu.make_async_copy(g_hbm.at[iblks[0].at[0]], bufs[dsl],
                                  semg).wait()

        def wait_scat(dsl):
            pltpu.make_async_copy(zeros_hbm.at[pl.ds(0, _CKS)],
                                  bufs[dsl], sems).wait()

        def fire_gather(isl, q, dsl):
            pltpu.async_copy(g_hbm.at[iblks[isl].at[2 * q]], bufs[dsl], semg)

        def fire_scat(isl, q, dsl):
            pltpu.async_copy(bufs[dsl], acc.at[iblks[isl].at[2 * q + 1]],
                             sems, add=True)

        pltpu.sync_copy(combo_hbm.at[c, s, 0], iblks[0])
        fire_gather(0, 0, 0)
        pltpu.sync_copy(zeros_hbm.at[pl.ds(0, _CKS)], bufs[1])
        pltpu.async_copy(bufs[1], acc.at[iblks[0].at[1]], sems, add=True)

        for bb in range(nblk):
            isl = bb % 2

            def pair(j, carry, _isl=isl, _bb=bb):
                for k in range(2):
                    q = 2 * j + k
                    dsl = (_bb + k) % 2
                    wait_gather(dsl)
                    wait_scat(1 - dsl)
                    fire_gather(_isl, q + 1, 1 - dsl)
                    fire_scat(_isl, q, dsl)
                return carry

            lax.fori_loop(0, kpairs, pair, 0)
            dsl = bb % 2
            wait_gather(dsl)
            wait_scat(1 - dsl)
            if bb + 1 < nblk:
                pltpu.sync_copy(combo_hbm.at[c, s, bb + 1],
                                iblks[(bb + 1) % 2])
                fire_gather((bb + 1) % 2, 0, 1 - dsl)
                fire_scat(isl, kpb - 1, dsl)
            else:
                pltpu.sync_copy(bufs[dsl],
                                acc.at[iblks[isl].at[2 * (kpb - 1) + 1]],
                                add=True)
        plsc.subcore_barrier()
        pltpu.sync_copy(acc.at[pl.ds(s * rpt, rpt)],
                        out_hbm.at[c, pl.ds(s * rpt, rpt)])

    return scat_kernel


def _dot(a, b):
    return jnp.dot(a, b, precision=lax.Precision.HIGHEST,
                   preferred_element_type=jnp.float32)


def _split2(g):
    half = g.shape[1] // 2
    return jnp.concatenate([g[None, :, :half], g[None, :, half:]], axis=0)


def _tc_a1_body(x_ref, w_ref, h_ref):
    h_ref[...] = _dot(x_ref[...], w_ref[...])


@functools.lru_cache(maxsize=None)
def _tc_a1(n, d, h, blk):
    return pl.pallas_call(
        _tc_a1_body,
        grid=(n // blk,),
        in_specs=[
            pl.BlockSpec((blk, d), lambda b: (b, 0)),
            pl.BlockSpec((d, h), lambda b: (0, 0)),
        ],
        out_specs=pl.BlockSpec((blk, h), lambda b: (b, 0)),
        out_shape=jax.ShapeDtypeStruct((n, h), jnp.float32),
    )


def _tc_a2_body(deg_ref, h_ref, g_ref, dinv_ref):
    deg = deg_ref[0, :, :_LN] + deg_ref[1, :, :_LN] + 1.0
    dinv = lax.rsqrt(deg)
    g = h_ref[...] * dinv[:, 0:1]
    g_ref[...] = _split2(g)
    dinv_ref[...] = dinv


@functools.lru_cache(maxsize=None)
def _tc_a2(n, h, blk):
    return pl.pallas_call(
        _tc_a2_body,
        grid=(n // blk,),
        in_specs=[
            pl.BlockSpec((_NC, blk, h // 2), lambda b: (0, b, 0)),
            pl.BlockSpec((blk, h), lambda b: (b, 0)),
        ],
        out_specs=[
            pl.BlockSpec((2, blk, h // 2), lambda b: (0, b, 0)),
            pl.BlockSpec((blk, _LN), lambda b: (b, 0)),
        ],
        out_shape=[
            jax.ShapeDtypeStruct((2, n, h // 2), jnp.float32),
            jax.ShapeDtypeStruct((n, _LN), jnp.float32),
        ],
    )


def _tc_b_body(alo_ref, ahi_ref, h1_ref, dinv_ref, b1_ref, w2_ref,
               h2_ref, g_ref):
    d1 = dinv_ref[:, 0:1]
    acc = jnp.concatenate([alo_ref[...], ahi_ref[...]], axis=1)
    out1 = d1 * acc + (d1 * d1) * h1_ref[...] + b1_ref[...]
    m = jnp.maximum(out1, 0.0)
    h2 = _dot(m, w2_ref[...])
    g2 = h2 * d1
    h2_ref[...] = h2
    g_ref[...] = _split2(g2)


@functools.lru_cache(maxsize=None)
def _tc_b(n, h, blk):
    return pl.pallas_call(
        _tc_b_body,
        grid=(n // blk,),
        in_specs=[
            pl.BlockSpec((blk, h // 2), lambda b: (b, 0)),
            pl.BlockSpec((blk, h // 2), lambda b: (b, 0)),
            pl.BlockSpec((blk, h), lambda b: (b, 0)),
            pl.BlockSpec((blk, _LN), lambda b: (b, 0)),
            pl.BlockSpec((1, h), lambda b: (0, 0)),
            pl.BlockSpec((h, h), lambda b: (0, 0)),
        ],
        out_specs=[
            pl.BlockSpec((blk, h), lambda b: (b, 0)),
            pl.BlockSpec((2, blk, h // 2), lambda b: (0, b, 0)),
        ],
        out_shape=[
            jax.ShapeDtypeStruct((n, h), jnp.float32),
            jax.ShapeDtypeStruct((2, n, h // 2), jnp.float32),
        ],
    )


def _tc_c_body(alo_ref, ahi_ref, h2_ref, dinv_ref, b2_ref, wo_ref, bo_ref,
               out_ref):
    d1 = dinv_ref[:, 0:1]
    acc = jnp.concatenate([alo_ref[...], ahi_ref[...]], axis=1)
    out2 = d1 * acc + (d1 * d1) * h2_ref[...] + b2_ref[...]
    logits = _dot(out2, wo_ref[...]) + bo_ref[...]
    mx = jnp.max(logits, axis=1, keepdims=True)
    sh = logits - mx
    lse = jnp.log(jnp.sum(jnp.exp(sh), axis=1, keepdims=True))
    out_ref[...] = sh - lse


@functools.lru_cache(maxsize=None)
def _tc_c(n, h, cdim, blk):
    return pl.pallas_call(
        _tc_c_body,
        grid=(n // blk,),
        in_specs=[
            pl.BlockSpec((blk, h // 2), lambda b: (b, 0)),
            pl.BlockSpec((blk, h // 2), lambda b: (b, 0)),
            pl.BlockSpec((blk, h), lambda b: (b, 0)),
            pl.BlockSpec((blk, _LN), lambda b: (b, 0)),
            pl.BlockSpec((1, h), lambda b: (0, 0)),
            pl.BlockSpec((h, cdim), lambda b: (0, 0)),
            pl.BlockSpec((1, cdim), lambda b: (0, 0)),
        ],
        out_specs=pl.BlockSpec((blk, cdim), lambda b: (b, 0)),
        out_shape=jax.ShapeDtypeStruct((n, cdim), jnp.float32),
    )


def kernel(x, edge_index, W1, b1, W2, b2, Wo, bo):
    n, d = x.shape
    e = edge_index.shape[1]
    h = W1.shape[1]
    cdim = Wo.shape[1]
    f = h // 2
    blk = 2000

    npad = _pad_n(n)
    rows = edge_index[0]
    cols = edge_index[1]
    ones_d = jnp.ones((_CKD, f), jnp.float32)
    zeros_f = jnp.zeros((npad, f), jnp.float32)

    nchunks = (e // _NS) // _CKS
    rows_r = rows.reshape(_NS, nchunks, _CKS)
    cols_r = cols.reshape(_NS, nchunks, _CKS)
    combo = jnp.stack([jnp.stack([rows_r, cols_r], axis=2),
                       jnp.stack([rows_r + n, cols_r], axis=2)])
    combo = combo.reshape(2, _NS, _NBLK, 2 * (nchunks // _NBLK), _CKS)
    nchunks_d = (e // _NC // _NS) // _CKD
    cols_d = cols.reshape(_NC, _NS, nchunks_d, _CKD)

    h1 = _tc_a1(n, d, h, blk)(x, W1)
    degs = _degree_sc(n, e, f)(cols_d, zeros_f, ones_d)[:, :n]
    g1, dinv = _tc_a2(n, h, blk)(degs, h1)
    acc1 = _scatter_sc(n, e, f)(combo, g1.reshape(2 * n, f), zeros_f)
    h2, g2 = _tc_b(n, h, blk)(
        acc1[0, :n], acc1[1, :n], h1, dinv, b1.reshape(1, -1), W2)
    acc2 = _scatter_sc(n, e, f)(combo, g2.reshape(2 * n, f), zeros_f)
    return _tc_c(n, h, cdim, blk)(
        acc2[0, :n], acc2[1, :n], h2, dinv, b2.reshape(1, -1), Wo, bo.reshape(1, -1))


# speedup vs baseline: 12.9233x; 1.0093x over previous
import functools

import jax
import jax.numpy as jnp
from jax import lax
from jax.experimental import pallas as pl
from jax.experimental.pallas import tpu as pltpu
from jax.experimental.pallas import tpu_sc as plsc

_NC = 2
_NS = 16
_LN = 16
_CKD = 40
_CKS = 80
_NBLK = 5


def _pad_n(n):
    step = _NS * 8
    return ((n + step - 1) // step) * step


@functools.lru_cache(maxsize=None)
def _degree_sc(n, e, fw):
    epc = e // _NC
    ept = epc // _NS
    nchunks = ept // _CKD
    npad = _pad_n(n)
    rpt = npad // _NS

    mesh = plsc.VectorSubcoreMesh(
        core_axis_name="c", subcore_axis_name="s",
        num_cores=_NC, num_subcores=_NS)

    @functools.partial(
        pl.kernel,
        out_type=jax.ShapeDtypeStruct((_NC, npad, fw), jnp.float32),
        mesh=mesh,
        scratch_types=[
            pltpu.VMEM_SHARED((npad, fw), jnp.float32),
            pltpu.VMEM((_CKD, fw), jnp.float32),
            pltpu.VMEM((nchunks, _CKD), jnp.int32),
            pltpu.SemaphoreType.DMA,
        ],
    )
    def deg_kernel(colsd_hbm, zeros_hbm, ones_hbm, out_hbm, acc, ones_v, colpre,
                   semd):
        c = lax.axis_index("c")
        s = lax.axis_index("s")
        pltpu.sync_copy(zeros_hbm.at[pl.ds(s * rpt, rpt)],
                        acc.at[pl.ds(s * rpt, rpt)])
        pltpu.sync_copy(ones_hbm, ones_v)
        pltpu.sync_copy(colsd_hbm.at[c, s], colpre)
        plsc.subcore_barrier()

        win = 6

        def fire(i):
            pltpu.async_copy(ones_v, acc.at[colpre.at[i]], semd, add=True)

        def drain():
            pltpu.make_async_copy(zeros_hbm.at[pl.ds(0, _CKD)], ones_v,
                                  semd).wait()

        for i in range(win):
            fire(i)

        def body(i, carry):
            drain()
            fire(i + win)
            return carry

        lax.fori_loop(0, nchunks - win, body, 0)
        for _ in range(win):
            drain()
        plsc.subcore_barrier()
        pltpu.sync_copy(acc.at[pl.ds(s * rpt, rpt)],
                        out_hbm.at[c, pl.ds(s * rpt, rpt)])

    return deg_kernel


@functools.lru_cache(maxsize=None)
def _scatter_sc(n, e, f):
    ept = e // _NS
    nchunks = ept // _CKS
    npad = _pad_n(n)
    rpt = npad // _NS
    nblk = _NBLK
    kpb = nchunks // nblk
    kpairs = (kpb - 1) // 2

    mesh = plsc.VectorSubcoreMesh(
        core_axis_name="c", subcore_axis_name="s",
        num_cores=_NC, num_subcores=_NS)

    @functools.partial(
        pl.kernel,
        out_type=jax.ShapeDtypeStruct((_NC, npad, f), jnp.float32),
        mesh=mesh,
        scratch_types=[
            pltpu.VMEM_SHARED((npad, f), jnp.float32),
            pltpu.VMEM((_CKS, f), jnp.float32),
            pltpu.VMEM((_CKS, f), jnp.float32),
            pltpu.VMEM((2 * kpb, _CKS), jnp.int32),
            pltpu.VMEM((2 * kpb, _CKS), jnp.int32),
            pltpu.VMEM((2 * kpb, _CKS), jnp.int32),
            pltpu.SemaphoreType.DMA,
            pltpu.SemaphoreType.DMA,
            pltpu.SemaphoreType.DMA,
        ],
    )
    def scat_kernel(combo_hbm, g_hbm, zeros_hbm, out_hbm,
                    acc, b0, b1, i0, i1, i2, semg, sems, semi):
        bufs = (b0, b1)
        iblks = (i0, i1, i2)
        c = lax.axis_index("c")
        s = lax.axis_index("s")
        pltpu.sync_copy(zeros_hbm.at[pl.ds(s * rpt, rpt)],
                        acc.at[pl.ds(s * rpt, rpt)])
        plsc.subcore_barrier()

        def wait_gather(dsl):
            pltpu.make_async_copy(g_hbm.at[iblks[0].at[0]], bufs[dsl],
                                  semg).wait()

        def wait_scat(dsl):
            pltpu.make_async_copy(zeros_hbm.at[pl.ds(0, _CKS)],
                                  bufs[dsl], sems).wait()

        def fire_gather(isl, q, dsl):
            pltpu.async_copy(g_hbm.at[iblks[isl].at[2 * q]], bufs[dsl], semg)

        def fire_scat(isl, q, dsl):
            pltpu.async_copy(bufs[dsl], acc.at[iblks[isl].at[2 * q + 1]],
                             sems, add=True)

        pltpu.sync_copy(combo_hbm.at[c, s, 0], iblks[0])
        if nblk > 1:
            pltpu.async_copy(combo_hbm.at[c, s, 1], iblks[1], semi)
        fire_gather(0, 0, 0)
        pltpu.sync_copy(zeros_hbm.at[pl.ds(0, _CKS)], bufs[1])
        pltpu.async_copy(bufs[1], acc.at[iblks[0].at[1]], sems, add=True)

        for bb in range(nblk):
            isl = bb % 3

            def pair(j, carry, _isl=isl, _bb=bb):
                for k in range(2):
                    q = 2 * j + k
                    dsl = (_bb + k) % 2
                    wait_gather(dsl)
                    wait_scat(1 - dsl)
                    fire_gather(_isl, q + 1, 1 - dsl)
                    fire_scat(_isl, q, dsl)
                return carry

            lax.fori_loop(0, kpairs, pair, 0)
            dsl = bb % 2
            wait_gather(dsl)
            wait_scat(1 - dsl)
            if bb + 1 < nblk:
                pltpu.make_async_copy(combo_hbm.at[c, s, 0],
                                      iblks[(bb + 1) % 3], semi).wait()
                if bb + 2 < nblk:
                    pltpu.async_copy(combo_hbm.at[c, s, bb + 2],
                                     iblks[(bb + 2) % 3], semi)
                fire_gather((bb + 1) % 3, 0, 1 - dsl)
                fire_scat(isl, kpb - 1, dsl)
            else:
                pltpu.sync_copy(bufs[dsl],
                                acc.at[iblks[isl].at[2 * (kpb - 1) + 1]],
                                add=True)
        plsc.subcore_barrier()
        pltpu.sync_copy(acc.at[pl.ds(s * rpt, rpt)],
                        out_hbm.at[c, pl.ds(s * rpt, rpt)])

    return scat_kernel


def _dot(a, b):
    return jnp.dot(a, b, precision=lax.Precision.HIGHEST,
                   preferred_element_type=jnp.float32)


def _split2(g):
    half = g.shape[1] // 2
    return jnp.concatenate([g[None, :, :half], g[None, :, half:]], axis=0)


def _tc_a1_body(x_ref, w_ref, h_ref):
    h_ref[...] = _dot(x_ref[...], w_ref[...])


@functools.lru_cache(maxsize=None)
def _tc_a1(n, d, h, blk):
    return pl.pallas_call(
        _tc_a1_body,
        grid=(n // blk,),
        in_specs=[
            pl.BlockSpec((blk, d), lambda b: (b, 0)),
            pl.BlockSpec((d, h), lambda b: (0, 0)),
        ],
        out_specs=pl.BlockSpec((blk, h), lambda b: (b, 0)),
        out_shape=jax.ShapeDtypeStruct((n, h), jnp.float32),
    )


def _tc_a2_body(deg_ref, h_ref, g_ref, dinv_ref):
    deg = deg_ref[0, :, :_LN] + deg_ref[1, :, :_LN] + 1.0
    dinv = lax.rsqrt(deg)
    g = h_ref[...] * dinv[:, 0:1]
    g_ref[...] = _split2(g)
    dinv_ref[...] = dinv


@functools.lru_cache(maxsize=None)
def _tc_a2(n, h, blk):
    return pl.pallas_call(
        _tc_a2_body,
        grid=(n // blk,),
        in_specs=[
            pl.BlockSpec((_NC, blk, h // 2), lambda b: (0, b, 0)),
            pl.BlockSpec((blk, h), lambda b: (b, 0)),
        ],
        out_specs=[
            pl.BlockSpec((2, blk, h // 2), lambda b: (0, b, 0)),
            pl.BlockSpec((blk, _LN), lambda b: (b, 0)),
        ],
        out_shape=[
            jax.ShapeDtypeStruct((2, n, h // 2), jnp.float32),
            jax.ShapeDtypeStruct((n, _LN), jnp.float32),
        ],
    )


def _tc_b_body(alo_ref, ahi_ref, h1_ref, dinv_ref, b1_ref, w2_ref,
               h2_ref, g_ref):
    d1 = dinv_ref[:, 0:1]
    acc = jnp.concatenate([alo_ref[...], ahi_ref[...]], axis=1)
    out1 = d1 * acc + (d1 * d1) * h1_ref[...] + b1_ref[...]
    m = jnp.maximum(out1, 0.0)
    h2 = _dot(m, w2_ref[...])
    g2 = h2 * d1
    h2_ref[...] = h2
    g_ref[...] = _split2(g2)


@functools.lru_cache(maxsize=None)
def _tc_b(n, h, blk):
    return pl.pallas_call(
        _tc_b_body,
        grid=(n // blk,),
        in_specs=[
            pl.BlockSpec((blk, h // 2), lambda b: (b, 0)),
            pl.BlockSpec((blk, h // 2), lambda b: (b, 0)),
            pl.BlockSpec((blk, h), lambda b: (b, 0)),
            pl.BlockSpec((blk, _LN), lambda b: (b, 0)),
            pl.BlockSpec((1, h), lambda b: (0, 0)),
            pl.BlockSpec((h, h), lambda b: (0, 0)),
        ],
        out_specs=[
            pl.BlockSpec((blk, h), lambda b: (b, 0)),
            pl.BlockSpec((2, blk, h // 2), lambda b: (0, b, 0)),
        ],
        out_shape=[
            jax.ShapeDtypeStruct((n, h), jnp.float32),
            jax.ShapeDtypeStruct((2, n, h // 2), jnp.float32),
        ],
    )


def _tc_c_body(alo_ref, ahi_ref, h2_ref, dinv_ref, b2_ref, wo_ref, bo_ref,
               out_ref):
    d1 = dinv_ref[:, 0:1]
    acc = jnp.concatenate([alo_ref[...], ahi_ref[...]], axis=1)
    out2 = d1 * acc + (d1 * d1) * h2_ref[...] + b2_ref[...]
    logits = _dot(out2, wo_ref[...]) + bo_ref[...]
    mx = jnp.max(logits, axis=1, keepdims=True)
    sh = logits - mx
    lse = jnp.log(jnp.sum(jnp.exp(sh), axis=1, keepdims=True))
    out_ref[...] = sh - lse


@functools.lru_cache(maxsize=None)
def _tc_c(n, h, cdim, blk):
    return pl.pallas_call(
        _tc_c_body,
        grid=(n // blk,),
        in_specs=[
            pl.BlockSpec((blk, h // 2), lambda b: (b, 0)),
            pl.BlockSpec((blk, h // 2), lambda b: (b, 0)),
            pl.BlockSpec((blk, h), lambda b: (b, 0)),
            pl.BlockSpec((blk, _LN), lambda b: (b, 0)),
            pl.BlockSpec((1, h), lambda b: (0, 0)),
            pl.BlockSpec((h, cdim), lambda b: (0, 0)),
            pl.BlockSpec((1, cdim), lambda b: (0, 0)),
        ],
        out_specs=pl.BlockSpec((blk, cdim), lambda b: (b, 0)),
        out_shape=jax.ShapeDtypeStruct((n, cdim), jnp.float32),
    )


def kernel(x, edge_index, W1, b1, W2, b2, Wo, bo):
    n, d = x.shape
    e = edge_index.shape[1]
    h = W1.shape[1]
    cdim = Wo.shape[1]
    f = h // 2
    blk = 2000

    npad = _pad_n(n)
    rows = edge_index[0]
    cols = edge_index[1]
    ones_d = jnp.ones((_CKD, f), jnp.float32)
    zeros_f = jnp.zeros((npad, f), jnp.float32)

    nchunks = (e // _NS) // _CKS
    rows_r = rows.reshape(_NS, nchunks, _CKS)
    cols_r = cols.reshape(_NS, nchunks, _CKS)
    combo = jnp.stack([jnp.stack([rows_r, cols_r], axis=2),
                       jnp.stack([rows_r + n, cols_r], axis=2)])
    combo = combo.reshape(2, _NS, _NBLK, 2 * (nchunks // _NBLK), _CKS)
    nchunks_d = (e // _NC // _NS) // _CKD
    cols_d = cols.reshape(_NC, _NS, nchunks_d, _CKD)

    h1 = _tc_a1(n, d, h, blk)(x, W1)
    degs = _degree_sc(n, e, f)(cols_d, zeros_f, ones_d)[:, :n]
    g1, dinv = _tc_a2(n, h, blk)(degs, h1)
    acc1 = _scatter_sc(n, e, f)(combo, g1.reshape(2 * n, f), zeros_f)
    h2, g2 = _tc_b(n, h, blk)(
        acc1[0, :n], acc1[1, :n], h1, dinv, b1.reshape(1, -1), W2)
    acc2 = _scatter_sc(n, e, f)(combo, g2.reshape(2 * n, f), zeros_f)
    return _tc_c(n, h, cdim, blk)(
        acc2[0, :n], acc2[1, :n], h2, dinv, b2.reshape(1, -1), Wo, bo.reshape(1, -1))


# speedup vs baseline: 13.0758x; 1.0118x over previous
import functools

import jax
import jax.numpy as jnp
from jax import lax
from jax.experimental import pallas as pl
from jax.experimental.pallas import tpu as pltpu
from jax.experimental.pallas import tpu_sc as plsc

_NC = 2
_NS = 16
_LN = 16
_CKD = 40
_CKS = 80
_NBLK = 5


def _pad_n(n):
    step = _NS * 8
    return ((n + step - 1) // step) * step


@functools.lru_cache(maxsize=None)
def _degree_sc(n, e, fw):
    epc = e // _NC
    ept = epc // _NS
    nchunks = ept // _CKD
    npad = _pad_n(n)
    rpt = npad // _NS

    mesh = plsc.VectorSubcoreMesh(
        core_axis_name="c", subcore_axis_name="s",
        num_cores=_NC, num_subcores=_NS)

    @functools.partial(
        pl.kernel,
        out_type=jax.ShapeDtypeStruct((_NC, npad, fw), jnp.float32),
        mesh=mesh,
        scratch_types=[
            pltpu.VMEM_SHARED((npad, fw), jnp.float32),
            pltpu.VMEM((_CKD, fw), jnp.float32),
            pltpu.VMEM((nchunks, _CKD), jnp.int32),
            pltpu.SemaphoreType.DMA,
        ],
    )
    def deg_kernel(colsd_hbm, zeros_hbm, ones_hbm, out_hbm, acc, ones_v, colpre,
                   semd):
        c = lax.axis_index("c")
        s = lax.axis_index("s")
        pltpu.sync_copy(zeros_hbm.at[pl.ds(s * rpt, rpt)],
                        acc.at[pl.ds(s * rpt, rpt)])
        pltpu.sync_copy(ones_hbm, ones_v)
        pltpu.sync_copy(colsd_hbm.at[c, s], colpre)
        plsc.subcore_barrier()

        win = 6

        def fire(i):
            pltpu.async_copy(ones_v, acc.at[colpre.at[i]], semd, add=True)

        def drain():
            pltpu.make_async_copy(zeros_hbm.at[pl.ds(0, _CKD)], ones_v,
                                  semd).wait()

        for i in range(win):
            fire(i)

        def body(i, carry):
            drain()
            fire(i + win)
            return carry

        lax.fori_loop(0, nchunks - win, body, 0)
        for _ in range(win):
            drain()
        plsc.subcore_barrier()
        pltpu.sync_copy(acc.at[pl.ds(s * rpt, rpt)],
                        out_hbm.at[c, pl.ds(s * rpt, rpt)])

    return deg_kernel


@functools.lru_cache(maxsize=None)
def _scatter_sc(n, e, f):
    ept = e // _NS
    nchunks = ept // _CKS
    npad = _pad_n(n)
    rpt = npad // _NS
    nblk = _NBLK
    kpb = nchunks // nblk
    kpairs = (kpb - 1) // 2

    mesh = plsc.VectorSubcoreMesh(
        core_axis_name="c", subcore_axis_name="s",
        num_cores=_NC, num_subcores=_NS)

    @functools.partial(
        pl.kernel,
        out_type=jax.ShapeDtypeStruct((_NC, npad, f), jnp.float32),
        mesh=mesh,
        scratch_types=[
            pltpu.VMEM_SHARED((npad, f), jnp.float32),
            pltpu.VMEM((_CKS, f), jnp.float32),
            pltpu.VMEM((_CKS, f), jnp.float32),
            pltpu.VMEM((2 * kpb, _CKS), jnp.int32),
            pltpu.VMEM((2 * kpb, _CKS), jnp.int32),
            pltpu.VMEM((2 * kpb, _CKS), jnp.int32),
            pltpu.SemaphoreType.DMA,
            pltpu.SemaphoreType.DMA,
            pltpu.SemaphoreType.DMA,
        ],
    )
    def scat_kernel(combo_hbm, g_hbm, zeros_hbm, out_hbm,
                    acc, b0, b1, i0, i1, i2, semg, sems, semi):
        bufs = (b0, b1)
        iblks = (i0, i1, i2)
        c = lax.axis_index("c")
        s = lax.axis_index("s")
        pltpu.sync_copy(zeros_hbm.at[pl.ds(s * rpt, rpt)],
                        acc.at[pl.ds(s * rpt, rpt)])
        plsc.subcore_barrier()

        def wait_gather(dsl):
            pltpu.make_async_copy(g_hbm.at[iblks[0].at[0]], bufs[dsl],
                                  semg).wait()

        def wait_scat(dsl):
            pltpu.make_async_copy(zeros_hbm.at[pl.ds(0, _CKS)],
                                  bufs[dsl], sems).wait()

        def fire_gather(isl, q, dsl):
            pltpu.async_copy(g_hbm.at[iblks[isl].at[2 * q]], bufs[dsl], semg)

        def fire_scat(isl, q, dsl):
            pltpu.async_copy(bufs[dsl], acc.at[iblks[isl].at[2 * q + 1]],
                             sems, add=True)

        pltpu.sync_copy(combo_hbm.at[c, s, 0], iblks[0])
        if nblk > 1:
            pltpu.async_copy(combo_hbm.at[c, s, 1], iblks[1], semi)
        fire_gather(0, 0, 0)
        pltpu.sync_copy(zeros_hbm.at[pl.ds(0, _CKS)], bufs[1])
        pltpu.async_copy(bufs[1], acc.at[iblks[0].at[1]], sems, add=True)

        for bb in range(nblk):
            isl = bb % 3

            def pair(j, carry, _isl=isl, _bb=bb):
                for k in range(2):
                    q = 2 * j + k
                    dsl = (_bb + k) % 2
                    wait_gather(dsl)
                    wait_scat(1 - dsl)
                    fire_gather(_isl, q + 1, 1 - dsl)
                    fire_scat(_isl, q, dsl)
                return carry

            lax.fori_loop(0, kpairs, pair, 0)
            dsl = bb % 2
            wait_gather(dsl)
            wait_scat(1 - dsl)
            if bb + 1 < nblk:
                pltpu.make_async_copy(combo_hbm.at[c, s, 0],
                                      iblks[(bb + 1) % 3], semi).wait()
                if bb + 2 < nblk:
                    pltpu.async_copy(combo_hbm.at[c, s, bb + 2],
                                     iblks[(bb + 2) % 3], semi)
                fire_gather((bb + 1) % 3, 0, 1 - dsl)
                fire_scat(isl, kpb - 1, dsl)
            else:
                pltpu.sync_copy(bufs[dsl],
                                acc.at[iblks[isl].at[2 * (kpb - 1) + 1]],
                                add=True)
        plsc.subcore_barrier()
        pltpu.sync_copy(acc.at[pl.ds(s * rpt, rpt)],
                        out_hbm.at[c, pl.ds(s * rpt, rpt)])

    return scat_kernel


def _dot(a, b):
    return jnp.dot(a, b, preferred_element_type=jnp.float32)


def _split2(g):
    half = g.shape[1] // 2
    return jnp.concatenate([g[None, :, :half], g[None, :, half:]], axis=0)


def _tc_a1_body(x_ref, w_ref, h_ref):
    h_ref[...] = _dot(x_ref[...], w_ref[...])


@functools.lru_cache(maxsize=None)
def _tc_a1(n, d, h, blk):
    return pl.pallas_call(
        _tc_a1_body,
        grid=(n // blk,),
        in_specs=[
            pl.BlockSpec((blk, d), lambda b: (b, 0)),
            pl.BlockSpec((d, h), lambda b: (0, 0)),
        ],
        out_specs=pl.BlockSpec((blk, h), lambda b: (b, 0)),
        out_shape=jax.ShapeDtypeStruct((n, h), jnp.float32),
    )


def _tc_a2_body(deg_ref, h_ref, g_ref, dinv_ref):
    deg = deg_ref[0, :, :_LN] + deg_ref[1, :, :_LN] + 1.0
    dinv = lax.rsqrt(deg)
    g = h_ref[...] * dinv[:, 0:1]
    g_ref[...] = _split2(g)
    dinv_ref[...] = dinv


@functools.lru_cache(maxsize=None)
def _tc_a2(n, h, blk):
    return pl.pallas_call(
        _tc_a2_body,
        grid=(n // blk,),
        in_specs=[
            pl.BlockSpec((_NC, blk, h // 2), lambda b: (0, b, 0)),
            pl.BlockSpec((blk, h), lambda b: (b, 0)),
        ],
        out_specs=[
            pl.BlockSpec((2, blk, h // 2), lambda b: (0, b, 0)),
            pl.BlockSpec((blk, _LN), lambda b: (b, 0)),
        ],
        out_shape=[
            jax.ShapeDtypeStruct((2, n, h // 2), jnp.float32),
            jax.ShapeDtypeStruct((n, _LN), jnp.float32),
        ],
    )


def _tc_b_body(alo_ref, ahi_ref, h1_ref, dinv_ref, b1_ref, w2_ref,
               h2_ref, g_ref):
    d1 = dinv_ref[:, 0:1]
    acc = jnp.concatenate([alo_ref[...], ahi_ref[...]], axis=1)
    out1 = d1 * acc + (d1 * d1) * h1_ref[...] + b1_ref[...]
    m = jnp.maximum(out1, 0.0)
    h2 = _dot(m, w2_ref[...])
    g2 = h2 * d1
    h2_ref[...] = h2
    g_ref[...] = _split2(g2)


@functools.lru_cache(maxsize=None)
def _tc_b(n, h, blk):
    return pl.pallas_call(
        _tc_b_body,
        grid=(n // blk,),
        in_specs=[
            pl.BlockSpec((blk, h // 2), lambda b: (b, 0)),
            pl.BlockSpec((blk, h // 2), lambda b: (b, 0)),
            pl.BlockSpec((blk, h), lambda b: (b, 0)),
            pl.BlockSpec((blk, _LN), lambda b: (b, 0)),
            pl.BlockSpec((1, h), lambda b: (0, 0)),
            pl.BlockSpec((h, h), lambda b: (0, 0)),
        ],
        out_specs=[
            pl.BlockSpec((blk, h), lambda b: (b, 0)),
            pl.BlockSpec((2, blk, h // 2), lambda b: (0, b, 0)),
        ],
        out_shape=[
            jax.ShapeDtypeStruct((n, h), jnp.float32),
            jax.ShapeDtypeStruct((2, n, h // 2), jnp.float32),
        ],
    )


def _tc_c_body(alo_ref, ahi_ref, h2_ref, dinv_ref, b2_ref, wo_ref, bo_ref,
               out_ref):
    d1 = dinv_ref[:, 0:1]
    acc = jnp.concatenate([alo_ref[...], ahi_ref[...]], axis=1)
    out2 = d1 * acc + (d1 * d1) * h2_ref[...] + b2_ref[...]
    logits = _dot(out2, wo_ref[...]) + bo_ref[...]
    mx = jnp.max(logits, axis=1, keepdims=True)
    sh = logits - mx
    lse = jnp.log(jnp.sum(jnp.exp(sh), axis=1, keepdims=True))
    out_ref[...] = sh - lse


@functools.lru_cache(maxsize=None)
def _tc_c(n, h, cdim, blk):
    return pl.pallas_call(
        _tc_c_body,
        grid=(n // blk,),
        in_specs=[
            pl.BlockSpec((blk, h // 2), lambda b: (b, 0)),
            pl.BlockSpec((blk, h // 2), lambda b: (b, 0)),
            pl.BlockSpec((blk, h), lambda b: (b, 0)),
            pl.BlockSpec((blk, _LN), lambda b: (b, 0)),
            pl.BlockSpec((1, h), lambda b: (0, 0)),
            pl.BlockSpec((h, cdim), lambda b: (0, 0)),
            pl.BlockSpec((1, cdim), lambda b: (0, 0)),
        ],
        out_specs=pl.BlockSpec((blk, cdim), lambda b: (b, 0)),
        out_shape=jax.ShapeDtypeStruct((n, cdim), jnp.float32),
    )


def kernel(x, edge_index, W1, b1, W2, b2, Wo, bo):
    n, d = x.shape
    e = edge_index.shape[1]
    h = W1.shape[1]
    cdim = Wo.shape[1]
    f = h // 2
    blk = 2000

    npad = _pad_n(n)
    rows = edge_index[0]
    cols = edge_index[1]
    ones_d = jnp.ones((_CKD, f), jnp.float32)
    zeros_f = jnp.zeros((npad, f), jnp.float32)

    nchunks = (e // _NS) // _CKS
    rows_r = rows.reshape(_NS, nchunks, _CKS)
    cols_r = cols.reshape(_NS, nchunks, _CKS)
    combo = jnp.stack([jnp.stack([rows_r, cols_r], axis=2),
                       jnp.stack([rows_r + n, cols_r], axis=2)])
    combo = combo.reshape(2, _NS, _NBLK, 2 * (nchunks // _NBLK), _CKS)
    nchunks_d = (e // _NC // _NS) // _CKD
    cols_d = cols.reshape(_NC, _NS, nchunks_d, _CKD)

    h1 = _tc_a1(n, d, h, blk)(x, W1)
    degs = _degree_sc(n, e, f)(cols_d, zeros_f, ones_d)[:, :n]
    g1, dinv = _tc_a2(n, h, blk)(degs, h1)
    acc1 = _scatter_sc(n, e, f)(combo, g1.reshape(2 * n, f), zeros_f)
    h2, g2 = _tc_b(n, h, blk)(
        acc1[0, :n], acc1[1, :n], h1, dinv, b1.reshape(1, -1), W2)
    acc2 = _scatter_sc(n, e, f)(combo, g2.reshape(2 * n, f), zeros_f)
    return _tc_c(n, h, cdim, blk)(
        acc2[0, :n], acc2[1, :n], h2, dinv, b2.reshape(1, -1), Wo, bo.reshape(1, -1))
